# Initial kernel scaffold; baseline (speedup 1.0000x reference)
import functools

import jax
import jax.numpy as jnp
from jax import lax
from jax.experimental import pallas as pl
from jax.experimental.pallas import tpu as pltpu
from jax.experimental.pallas import tpu_sc as plsc

NC, NS, LANES = 2, 16, 16
NW = NC * NS

_LN2 = 0.6931471805599453
_CUTOFF = 5.0
_NG = 50


def _ssp(x):
    return jax.nn.softplus(x) - _LN2


def _sc_mesh():
    return plsc.VectorSubcoreMesh(core_axis_name="c", subcore_axis_name="s")


def _sc_gather(tbl, idx, bsz):
    rows = idx.shape[0]
    d = tbl.shape[1]
    nblk = rows // (NW * bsz)

    @functools.partial(
        pl.kernel,
        out_type=jax.ShapeDtypeStruct((rows, d), jnp.float32),
        mesh=_sc_mesh(),
        scratch_types=[
            pltpu.VMEM((bsz,), jnp.int32),
            pltpu.VMEM((bsz, d), jnp.float32),
            pltpu.SemaphoreType.DMA,
        ],
    )
    def k(tbl_ref, idx_ref, out_ref, idx_v, rows_v, sem):
        wid = lax.axis_index("s") * NC + lax.axis_index("c")

        def body(i, c):
            base = (wid * nblk + i) * bsz
            pltpu.sync_copy(idx_ref.at[pl.ds(base, bsz)], idx_v)
            pltpu.async_copy(tbl_ref.at[idx_v], rows_v, sem).wait()
            pltpu.sync_copy(rows_v, out_ref.at[pl.ds(base, bsz)])
            return c

        lax.fori_loop(0, nblk, body, 0)

    return k(tbl, idx)


def _sc_dvec(xyz_pad, a0, a1):
    rows = a0.shape[0]
    d = xyz_pad.shape[1]
    bsz = 128
    nblk = rows // (NW * bsz)

    @functools.partial(
        pl.kernel,
        out_type=jax.ShapeDtypeStruct((rows, d), jnp.float32),
        mesh=_sc_mesh(),
        scratch_types=[
            pltpu.VMEM((bsz,), jnp.int32),
            pltpu.VMEM((bsz,), jnp.int32),
            pltpu.VMEM((bsz, d), jnp.float32),
            pltpu.VMEM((bsz, d), jnp.float32),
            pltpu.SemaphoreType.DMA,
            pltpu.SemaphoreType.DMA,
        ],
    )
    def k(tbl_ref, a0_ref, a1_ref, out_ref, i0_v, i1_v, r0_v, r1_v, s0, s1):
        wid = lax.axis_index("s") * NC + lax.axis_index("c")

        def body(i, c):
            base = (wid * nblk + i) * bsz
            pltpu.sync_copy(a0_ref.at[pl.ds(base, bsz)], i0_v)
            pltpu.sync_copy(a1_ref.at[pl.ds(base, bsz)], i1_v)
            c0 = pltpu.async_copy(tbl_ref.at[i0_v], r0_v, s0)
            c1 = pltpu.async_copy(tbl_ref.at[i1_v], r1_v, s1)
            c0.wait()
            c1.wait()

            def row(j, cc):
                r0_v[j] = r0_v[j] - r1_v[j]
                return cc

            lax.fori_loop(0, bsz, row, 0)
            pltpu.sync_copy(r0_v, out_ref.at[pl.ds(base, bsz)])
            return c

        lax.fori_loop(0, nblk, body, 0)

    return k(xyz_pad, a0, a1)


def _sc_convmsg(rf, ef, a0, a1, npad):
    rows = a0.shape[0]
    bsz = 128
    nblk = rows // (NW * bsz)
    rps = npad // NS
    nchunk = rps // bsz

    @functools.partial(
        pl.kernel,
        out_type=(
            jax.ShapeDtypeStruct((rows, 128), jnp.float32),
            jax.ShapeDtypeStruct((2, npad, 128), jnp.float32),
        ),
        mesh=_sc_mesh(),
        scratch_types=[
            pltpu.VMEM((bsz,), jnp.int32),
            pltpu.VMEM((bsz,), jnp.int32),
            pltpu.VMEM((bsz, 128), jnp.float32),
            pltpu.VMEM((bsz, 128), jnp.float32),
            pltpu.VMEM((bsz, 128), jnp.float32),
            pltpu.VMEM((bsz, 128), jnp.float32),
            pltpu.VMEM_SHARED((npad, 128), jnp.float32),
            pltpu.SemaphoreType.DMA,
            pltpu.SemaphoreType.DMA,
        ],
    )
    def k(rf_ref, ef_ref, a0_ref, a1_ref, s_out, drp_out,
          i0_v, i1_v, ef_v, r0_v, r1_v, s_v, acc, sem0, sem1):
        cid = lax.axis_index("c")
        sid = lax.axis_index("s")
        wid = sid * NC + cid

        def zrow(j, c):
            for kk in range(8):
                s_v[j, pl.ds(kk * 16, 16)] = jnp.zeros((16,), jnp.float32)
            return c

        lax.fori_loop(0, bsz, zrow, 0)

        def zc(q, c):
            pltpu.sync_copy(s_v, acc.at[pl.ds(sid * rps + q * bsz, bsz)])
            return c

        lax.fori_loop(0, nchunk, zc, 0)
        plsc.subcore_barrier()

        def body(i, c):
            base = (wid * nblk + i) * bsz
            pltpu.sync_copy(a0_ref.at[pl.ds(base, bsz)], i0_v)
            pltpu.sync_copy(a1_ref.at[pl.ds(base, bsz)], i1_v)
            c0 = pltpu.async_copy(rf_ref.at[i0_v], r0_v, sem0)
            c1 = pltpu.async_copy(rf_ref.at[i1_v], r1_v, sem1)
            pltpu.sync_copy(ef_ref.at[pl.ds(base, bsz)], ef_v)
            c0.wait()
            c1.wait()

            def row(j, cc):
                for kk in range(8):
                    dsl = pl.ds(kk * 16, 16)
                    e = ef_v[j, dsl]
                    x0 = r0_v[j, dsl] * e
                    x1 = r1_v[j, dsl] * e
                    r0_v[j, dsl] = x0
                    r1_v[j, dsl] = x1
                    s_v[j, dsl] = x0 + x1
                return cc

            lax.fori_loop(0, bsz, row, 0)
            pltpu.sync_copy(s_v, s_out.at[pl.ds(base, bsz)])
            pltpu.sync_copy(r0_v, acc.at[i1_v], add=True)
            pltpu.sync_copy(r1_v, acc.at[i0_v], add=True)
            return c

        lax.fori_loop(0, nblk, body, 0)
        plsc.subcore_barrier()

        def dump(q, c):
            ro = sid * rps + q * bsz
            pltpu.sync_copy(acc.at[pl.ds(ro, bsz)], s_v)
            pltpu.sync_copy(s_v, drp_out.at[cid, pl.ds(ro, bsz)])
            return c

        lax.fori_loop(0, nchunk, dump, 0)

    return k(rf, ef, a0, a1)


def _sc_fscatter(fpos, fneg, a0, a1, npad):
    rows = a0.shape[0]
    d = fpos.shape[1]
    bsz = 128
    nblk = rows // (NW * bsz)
    rps = npad // NS
    nchunk = rps // bsz

    @functools.partial(
        pl.kernel,
        out_type=jax.ShapeDtypeStruct((2, npad, d), jnp.float32),
        mesh=_sc_mesh(),
        scratch_types=[
            pltpu.VMEM((bsz,), jnp.int32),
            pltpu.VMEM((bsz,), jnp.int32),
            pltpu.VMEM((bsz, d), jnp.float32),
            pltpu.VMEM((bsz, d), jnp.float32),
            pltpu.VMEM_SHARED((npad, d), jnp.float32),
        ],
    )
    def k(fp_ref, fn_ref, a0_ref, a1_ref, out_ref,
          i0_v, i1_v, f0_v, f1_v, acc):
        cid = lax.axis_index("c")
        sid = lax.axis_index("s")
        wid = sid * NC + cid

        def zrow(j, c):
            f0_v[j] = jnp.zeros((d,), jnp.float32)
            return c

        lax.fori_loop(0, bsz, zrow, 0)

        def zc(q, c):
            pltpu.sync_copy(f0_v, acc.at[pl.ds(sid * rps + q * bsz, bsz)])
            return c

        lax.fori_loop(0, nchunk, zc, 0)
        plsc.subcore_barrier()

        def body(i, c):
            base = (wid * nblk + i) * bsz
            pltpu.sync_copy(a0_ref.at[pl.ds(base, bsz)], i0_v)
            pltpu.sync_copy(a1_ref.at[pl.ds(base, bsz)], i1_v)
            pltpu.sync_copy(fp_ref.at[pl.ds(base, bsz)], f0_v)
            pltpu.sync_copy(fn_ref.at[pl.ds(base, bsz)], f1_v)
            pltpu.sync_copy(f0_v, acc.at[i0_v], add=True)
            pltpu.sync_copy(f1_v, acc.at[i1_v], add=True)
            return c

        lax.fori_loop(0, nblk, body, 0)
        plsc.subcore_barrier()

        def dump(q, c):
            ro = sid * rps + q * bsz
            pltpu.sync_copy(acc.at[pl.ds(ro, bsz)], f0_v)
            pltpu.sync_copy(f0_v, out_ref.at[cid, pl.ds(ro, bsz)])
            return c

        lax.fori_loop(0, nchunk, dump, 0)

    return k(fpos, fneg, a0, a1)


_BLK = 512


def _row_spec(blk, d):
    return pl.BlockSpec((blk, d), lambda i: (i, 0))


def _full_spec(shape):
    nd = len(shape)
    return pl.BlockSpec(shape, lambda i: (0,) * nd)


def _tc_params():
    return pltpu.CompilerParams(dimension_semantics=("parallel",))


def _tc_geom(dvec, w1, b1, w2, b2, offs, width):
    rows, dcoord = dvec.shape
    grid = rows // _BLK

    def body(dv_ref, offs_ref, w1_ref, b1_ref, w2_ref, b2_ref, e_ref, adj_ref):
        dv = dv_ref[...]
        d2 = jnp.sum(dv * dv, axis=1, keepdims=True)
        dis = jnp.sqrt(d2)
        adj_ref[...] = dv / dis
        x = (dis - offs_ref[...]) / width
        e0 = jnp.exp(-0.5 * x * x)
        h = _ssp(jnp.dot(e0, w1_ref[...], preferred_element_type=jnp.float32)
                 + b1_ref[...])
        e_ref[...] = (jnp.dot(h, w2_ref[...], preferred_element_type=jnp.float32)
                      + b2_ref[...])

    return pl.pallas_call(
        body,
        grid=(grid,),
        in_specs=[
            _row_spec(_BLK, dcoord),
            _full_spec(offs.shape),
            _full_spec(w1.shape),
            _full_spec(b1.shape),
            _full_spec(w2.shape),
            _full_spec(b2.shape),
        ],
        out_specs=[_row_spec(_BLK, 128), _row_spec(_BLK, dcoord)],
        out_shape=[
            jax.ShapeDtypeStruct((rows, 128), jnp.float32),
            jax.ShapeDtypeStruct((rows, dcoord), jnp.float32),
        ],
        compiler_params=_tc_params(),
    )(dvec, offs, w1, b1, w2, b2)


def _tc_mlp(x, p1, p2, res=None):
    rows = x.shape[0]
    grid = rows // _BLK
    w1, b1 = p1[0], p1[1].reshape(1, -1)
    w2, b2 = p2[0], p2[1].reshape(1, -1)
    dout = w2.shape[1]

    def body(*refs):
        if res is None:
            x_ref, w1_ref, b1_ref, w2_ref, b2_ref, o_ref = refs
        else:
            x_ref, w1_ref, b1_ref, w2_ref, b2_ref, r_ref, o_ref = refs
        h = _ssp(jnp.dot(x_ref[...], w1_ref[...],
                         preferred_element_type=jnp.float32) + b1_ref[...])
        o = (jnp.dot(h, w2_ref[...], preferred_element_type=jnp.float32)
             + b2_ref[...])
        if res is not None:
            o = o + r_ref[...]
        o_ref[...] = o

    in_specs = [
        _row_spec(_BLK, x.shape[1]),
        _full_spec(w1.shape),
        _full_spec(b1.shape),
        _full_spec(w2.shape),
        _full_spec(b2.shape),
    ]
    args = [x, w1, b1, w2, b2]
    if res is not None:
        in_specs.append(_row_spec(_BLK, dout))
        args.append(res)

    return pl.pallas_call(
        body,
        grid=(grid,),
        in_specs=in_specs,
        out_specs=_row_spec(_BLK, dout),
        out_shape=jax.ShapeDtypeStruct((rows, dout), jnp.float32),
        compiler_params=_tc_params(),
    )(*args)


def _tc_mlp_drp(drp, p1, p2, res):
    rows = drp.shape[1]
    grid = rows // _BLK
    w1, b1 = p1[0], p1[1].reshape(1, -1)
    w2, b2 = p2[0], p2[1].reshape(1, -1)

    def body(d_ref, w1_ref, b1_ref, w2_ref, b2_ref, r_ref, o_ref):
        x = d_ref[0] + d_ref[1]
        h = _ssp(jnp.dot(x, w1_ref[...],
                         preferred_element_type=jnp.float32) + b1_ref[...])
        o_ref[...] = (jnp.dot(h, w2_ref[...],
                              preferred_element_type=jnp.float32)
                      + b2_ref[...] + r_ref[...])

    return pl.pallas_call(
        body,
        grid=(grid,),
        in_specs=[
            pl.BlockSpec((2, _BLK, 128), lambda i: (0, i, 0)),
            _full_spec(w1.shape),
            _full_spec(b1.shape),
            _full_spec(w2.shape),
            _full_spec(b2.shape),
            _row_spec(_BLK, 128),
        ],
        out_specs=_row_spec(_BLK, 128),
        out_shape=jax.ShapeDtypeStruct((rows, 128), jnp.float32),
        compiler_params=_tc_params(),
    )(drp, w1, b1, w2, b2, res)


def _tc_readout(e, adj, p1, p2):
    rows, dcoord = adj.shape
    grid = rows // _BLK
    w1, b1 = p1[0], p1[1].reshape(1, -1)
    w2, b2 = p2[0], p2[1].reshape(1, -1)

    def body(e_ref, a_ref, w1_ref, b1_ref, w2_ref, b2_ref, fp_ref, fn_ref):
        h = _ssp(jnp.dot(e_ref[...], w1_ref[...],
                         preferred_element_type=jnp.float32) + b1_ref[...])
        val = (jnp.dot(h, w2_ref[...], preferred_element_type=jnp.float32)
               + b2_ref[...])
        fp = val * a_ref[...]
        fp_ref[...] = fp
        fn_ref[...] = -fp

    return pl.pallas_call(
        body,
        grid=(grid,),
        in_specs=[
            _row_spec(_BLK, 128),
            _row_spec(_BLK, dcoord),
            _full_spec(w1.shape),
            _full_spec(b1.shape),
            _full_spec(w2.shape),
            _full_spec(b2.shape),
        ],
        out_specs=[_row_spec(_BLK, dcoord), _row_spec(_BLK, dcoord)],
        out_shape=[
            jax.ShapeDtypeStruct((rows, dcoord), jnp.float32),
            jax.ShapeDtypeStruct((rows, dcoord), jnp.float32),
        ],
        compiler_params=_tc_params(),
    )(e, adj, w1, b1, w2, b2)


def _tc_combine(facc):
    rows, d = facc.shape[1], facc.shape[2]
    grid = rows // _BLK

    def body(f_ref, o_ref):
        o_ref[...] = f_ref[0] + f_ref[1]

    return pl.pallas_call(
        body,
        grid=(grid,),
        in_specs=[pl.BlockSpec((2, _BLK, d), lambda i: (0, i, 0))],
        out_specs=_row_spec(_BLK, d),
        out_shape=jax.ShapeDtypeStruct((rows, d), jnp.float32),
        compiler_params=_tc_params(),
    )(facc)


def kernel(nxyz, nbr_list, params):
    n = nxyz.shape[0]
    e_cnt = nbr_list.shape[0]
    npad = -(-(n + 1) // 2048) * 2048
    nblk = -(-e_cnt // (NW * 128))
    epad = NW * 128 * nblk

    z = nxyz[:, 0].astype(jnp.int32)
    xyz = nxyz[:, 1:4].astype(jnp.float32)
    xyz_pad = jnp.zeros((npad, 16), jnp.float32).at[:n, :3].set(xyz)

    a = nbr_list.astype(jnp.int32)
    a0 = jnp.full((epad,), n, jnp.int32).at[:e_cnt].set(a[:, 0])
    a1 = jnp.full((epad,), n, jnp.int32).at[:e_cnt].set(a[:, 1])
    z_pad = jnp.zeros((npad,), jnp.int32).at[:n].set(z)

    emb = params['emb']
    emb_pad = jnp.zeros((128, emb.shape[1]), jnp.float32).at[:emb.shape[0]].set(emb)

    r = _sc_gather(emb_pad, z_pad, 64)
    dvec = _sc_dvec(xyz_pad, a0, a1)

    offs = jnp.linspace(0.0, _CUTOFF, _NG).astype(jnp.float32)
    width = float(_CUTOFF / (_NG - 1))
    offs = jnp.concatenate([offs, jnp.full((14,), 1e9, jnp.float32)])
    offs = offs.reshape(1, 64)

    efp = params['ef']
    w1 = jnp.zeros((64, efp[0][0].shape[1]), jnp.float32).at[:_NG].set(efp[0][0])
    e, adj = _tc_geom(dvec, w1, efp[0][1].reshape(1, -1),
                      efp[1][0], efp[1][1].reshape(1, -1), offs, width)

    for cp in params['convs']:
        ef = _tc_mlp(e, cp['edge_filter'][0], cp['edge_filter'][1])
        rf = _tc_mlp(r, cp['atom_filter'][0], cp['atom_filter'][1])
        s, drp = _sc_convmsg(rf, ef, a0, a1, npad)
        r = _tc_mlp_drp(drp, cp['atom_update'][0], cp['atom_update'][1], r)
        e = _tc_mlp(s, cp['edge_update'][0], cp['edge_update'][1], res=e)

    fpos, fneg = _tc_readout(e, adj, params['ro'][0], params['ro'][1])
    facc = _sc_fscatter(fpos, fneg, a0, a1, npad)
    f_atom = _tc_combine(facc)
    return f_atom[:n, :3]


# speedup vs baseline: 1.1407x; 1.1407x over previous
import functools

import jax
import jax.numpy as jnp
from jax import lax
from jax.experimental import pallas as pl
from jax.experimental.pallas import tpu as pltpu
from jax.experimental.pallas import tpu_sc as plsc

NC, NS, LANES = 2, 16, 16
NW = NC * NS

_LN2 = 0.6931471805599453
_CUTOFF = 5.0
_NG = 50
_HW = 64


def _ssp(x):
    return jax.nn.softplus(x) - _LN2


def _sc_mesh():
    return plsc.VectorSubcoreMesh(core_axis_name="c", subcore_axis_name="s")


_SC_CP = pltpu.CompilerParams(use_tc_tiling_on_sc=False)


def _sc_gather(tbl, idx, bsz):
    rows = idx.shape[0]
    d = tbl.shape[1]
    nblk = rows // (NW * bsz)

    @functools.partial(
        pl.kernel,
        out_type=pltpu.HBM((rows, d), jnp.float32),
        mesh=_sc_mesh(),
        scratch_types=[
            pltpu.VMEM((bsz,), jnp.int32),
            pltpu.VMEM((bsz, d), jnp.float32),
            pltpu.SemaphoreType.DMA,
        ],
    )
    def k(tbl_ref, idx_ref, out_ref, idx_v, rows_v, sem):
        wid = lax.axis_index("s") * NC + lax.axis_index("c")

        def body(i, c):
            base = (wid * nblk + i) * bsz
            pltpu.sync_copy(idx_ref.at[pl.ds(base, bsz)], idx_v)
            pltpu.async_copy(tbl_ref.at[idx_v], rows_v, sem).wait()
            pltpu.sync_copy(rows_v, out_ref.at[pl.ds(base, bsz)])
            return c

        lax.fori_loop(0, nblk, body, 0)

    return k(tbl, idx)


def _sc_dvec(xyz_pad, a0, a1):
    rows = a0.shape[0]
    d = xyz_pad.shape[1]
    bsz = 128
    nblk = rows // (NW * bsz)

    @functools.partial(
        pl.kernel,
        out_type=pltpu.HBM((rows, d), jnp.float32),
        mesh=_sc_mesh(),
        compiler_params=_SC_CP,
        scratch_types=[
            pltpu.VMEM((bsz,), jnp.int32),
            pltpu.VMEM((bsz,), jnp.int32),
            pltpu.VMEM((bsz, d), jnp.float32),
            pltpu.VMEM((bsz, d), jnp.float32),
            pltpu.SemaphoreType.DMA,
            pltpu.SemaphoreType.DMA,
        ],
    )
    def k(tbl_ref, a0_ref, a1_ref, out_ref, i0_v, i1_v, r0_v, r1_v, s0, s1):
        wid = lax.axis_index("s") * NC + lax.axis_index("c")

        def body(i, c):
            base = (wid * nblk + i) * bsz
            pltpu.sync_copy(a0_ref.at[pl.ds(base, bsz)], i0_v)
            pltpu.sync_copy(a1_ref.at[pl.ds(base, bsz)], i1_v)
            c0 = pltpu.async_copy(tbl_ref.at[i0_v], r0_v, s0)
            c1 = pltpu.async_copy(tbl_ref.at[i1_v], r1_v, s1)
            c0.wait()
            c1.wait()

            def row(j, cc):
                r0_v[j] = r0_v[j] - r1_v[j]
                return cc

            lax.fori_loop(0, bsz, row, 0)
            pltpu.sync_copy(r0_v, out_ref.at[pl.ds(base, bsz)])
            return c

        lax.fori_loop(0, nblk, body, 0)

    return k(xyz_pad, a0, a1)


def _sc_convmsg(rf2, ef2, a0, a1, npad):
    rows = a0.shape[0]
    bsz = 128
    hw = _HW
    nblk = rows // (NW * bsz)
    rps = npad // NS
    nchunk = rps // bsz

    @functools.partial(
        pl.kernel,
        out_type=(
            pltpu.HBM((2, rows, hw), jnp.float32),
            pltpu.HBM((2, 2, npad, hw), jnp.float32),
        ),
        mesh=_sc_mesh(),
        compiler_params=_SC_CP,
        scratch_types=[
            pltpu.VMEM((bsz,), jnp.int32),
            pltpu.VMEM((bsz,), jnp.int32),
            pltpu.VMEM((bsz, hw), jnp.float32),
            pltpu.VMEM((bsz, hw), jnp.float32),
            pltpu.VMEM((bsz, hw), jnp.float32),
            pltpu.VMEM((bsz, hw), jnp.float32),
            pltpu.VMEM_SHARED((npad, hw), jnp.float32),
            pltpu.SemaphoreType.DMA,
            pltpu.SemaphoreType.DMA,
        ],
    )
    def k(rf_ref, ef_ref, a0_ref, a1_ref, s_out, drp_out,
          i0_v, i1_v, ef_v, r0_v, r1_v, s_v, acc, sem0, sem1):
        cid = lax.axis_index("c")
        sid = lax.axis_index("s")
        wid = sid * NC + cid

        def zero_sv(j, c):
            for kk in range(hw // 16):
                s_v[j, pl.ds(kk * 16, 16)] = jnp.zeros((16,), jnp.float32)
            return c

        for h in range(2):
            lax.fori_loop(0, bsz, zero_sv, 0)

            def zc(q, c):
                pltpu.sync_copy(s_v, acc.at[pl.ds(sid * rps + q * bsz, bsz)])
                return c

            lax.fori_loop(0, nchunk, zc, 0)
            plsc.subcore_barrier()

            def body(i, c):
                base = (wid * nblk + i) * bsz
                pltpu.sync_copy(a0_ref.at[pl.ds(base, bsz)], i0_v)
                pltpu.sync_copy(a1_ref.at[pl.ds(base, bsz)], i1_v)
                c0 = pltpu.async_copy(rf_ref.at[h].at[i0_v], r0_v, sem0)
                c1 = pltpu.async_copy(rf_ref.at[h].at[i1_v], r1_v, sem1)
                pltpu.sync_copy(ef_ref.at[h, pl.ds(base, bsz)], ef_v)
                c0.wait()
                c1.wait()

                def row(j, cc):
                    for kk in range(hw // 16):
                        dsl = pl.ds(kk * 16, 16)
                        e = ef_v[j, dsl]
                        x0 = r0_v[j, dsl] * e
                        x1 = r1_v[j, dsl] * e
                        r0_v[j, dsl] = x0
                        r1_v[j, dsl] = x1
                        s_v[j, dsl] = x0 + x1
                    return cc

                lax.fori_loop(0, bsz, row, 0)
                pltpu.sync_copy(s_v, s_out.at[h, pl.ds(base, bsz)])
                pltpu.sync_copy(r0_v, acc.at[i1_v], add=True)
                pltpu.sync_copy(r1_v, acc.at[i0_v], add=True)
                return c

            lax.fori_loop(0, nblk, body, 0)
            plsc.subcore_barrier()

            def dump(q, c):
                ro = sid * rps + q * bsz
                pltpu.sync_copy(acc.at[pl.ds(ro, bsz)], s_v)
                pltpu.sync_copy(s_v, drp_out.at[h, cid, pl.ds(ro, bsz)])
                return c

            lax.fori_loop(0, nchunk, dump, 0)
            plsc.subcore_barrier()

    return k(rf2, ef2, a0, a1)


def _sc_fscatter(fpos, fneg, a0, a1, npad):
    rows = a0.shape[0]
    d = fpos.shape[1]
    bsz = 128
    nblk = rows // (NW * bsz)
    rps = npad // NS
    nchunk = rps // bsz

    @functools.partial(
        pl.kernel,
        out_type=pltpu.HBM((2, npad, d), jnp.float32),
        mesh=_sc_mesh(),
        compiler_params=_SC_CP,
        scratch_types=[
            pltpu.VMEM((bsz,), jnp.int32),
            pltpu.VMEM((bsz,), jnp.int32),
            pltpu.VMEM((bsz, d), jnp.float32),
            pltpu.VMEM((bsz, d), jnp.float32),
            pltpu.VMEM_SHARED((npad, d), jnp.float32),
        ],
    )
    def k(fp_ref, fn_ref, a0_ref, a1_ref, out_ref,
          i0_v, i1_v, f0_v, f1_v, acc):
        cid = lax.axis_index("c")
        sid = lax.axis_index("s")
        wid = sid * NC + cid

        def zrow(j, c):
            f0_v[j] = jnp.zeros((d,), jnp.float32)
            return c

        lax.fori_loop(0, bsz, zrow, 0)

        def zc(q, c):
            pltpu.sync_copy(f0_v, acc.at[pl.ds(sid * rps + q * bsz, bsz)])
            return c

        lax.fori_loop(0, nchunk, zc, 0)
        plsc.subcore_barrier()

        def body(i, c):
            base = (wid * nblk + i) * bsz
            pltpu.sync_copy(a0_ref.at[pl.ds(base, bsz)], i0_v)
            pltpu.sync_copy(a1_ref.at[pl.ds(base, bsz)], i1_v)
            pltpu.sync_copy(fp_ref.at[pl.ds(base, bsz)], f0_v)
            pltpu.sync_copy(fn_ref.at[pl.ds(base, bsz)], f1_v)
            pltpu.sync_copy(f0_v, acc.at[i0_v], add=True)
            pltpu.sync_copy(f1_v, acc.at[i1_v], add=True)
            return c

        lax.fori_loop(0, nblk, body, 0)
        plsc.subcore_barrier()

        def dump(q, c):
            ro = sid * rps + q * bsz
            pltpu.sync_copy(acc.at[pl.ds(ro, bsz)], f0_v)
            pltpu.sync_copy(f0_v, out_ref.at[cid, pl.ds(ro, bsz)])
            return c

        lax.fori_loop(0, nchunk, dump, 0)

    return k(fpos, fneg, a0, a1)


_BLK = 512


def _row_spec(blk, d):
    return pl.BlockSpec((blk, d), lambda i: (i, 0))


def _full_spec(shape):
    nd = len(shape)
    return pl.BlockSpec(shape, lambda i: (0,) * nd)


def _tc_params():
    return pltpu.CompilerParams(dimension_semantics=("parallel",))


def _tc_geom(dvec, w1, b1, w2, b2, offs, width):
    rows, dcoord = dvec.shape
    grid = rows // _BLK

    def body(dv_ref, offs_ref, w1_ref, b1_ref, w2_ref, b2_ref, e_ref, adj_ref):
        dv = dv_ref[...]
        d2 = jnp.sum(dv * dv, axis=1, keepdims=True)
        dis = jnp.sqrt(d2)
        adj_ref[...] = dv / dis
        x = (dis - offs_ref[...]) / width
        e0 = jnp.exp(-0.5 * x * x)
        h = _ssp(jnp.dot(e0, w1_ref[...], preferred_element_type=jnp.float32)
                 + b1_ref[...])
        e_ref[...] = (jnp.dot(h, w2_ref[...], preferred_element_type=jnp.float32)
                      + b2_ref[...])

    return pl.pallas_call(
        body,
        grid=(grid,),
        in_specs=[
            _row_spec(_BLK, dcoord),
            _full_spec(offs.shape),
            _full_spec(w1.shape),
            _full_spec(b1.shape),
            _full_spec(w2.shape),
            _full_spec(b2.shape),
        ],
        out_specs=[_row_spec(_BLK, 128), _row_spec(_BLK, dcoord)],
        out_shape=[
            jax.ShapeDtypeStruct((rows, 128), jnp.float32),
            jax.ShapeDtypeStruct((rows, dcoord), jnp.float32),
        ],
        compiler_params=_tc_params(),
    )(dvec, offs, w1, b1, w2, b2)


def _tc_mlp(x, p1, p2, res=None, split_in=False, split_out=False,
            drp_in=False):
    if drp_in:
        rows = x.shape[2]
    elif split_in:
        rows = x.shape[1]
    else:
        rows = x.shape[0]
    grid = rows // _BLK
    w1, b1 = p1[0], p1[1].reshape(1, -1)
    w2, b2 = p2[0], p2[1].reshape(1, -1)
    dout = w2.shape[1]

    def body(*refs):
        refs = list(refs)
        x_ref = refs.pop(0)
        w1_ref, b1_ref, w2_ref, b2_ref = refs[:4]
        refs = refs[4:]
        r_ref = refs.pop(0) if res is not None else None
        o_ref = refs.pop(0)
        if drp_in:
            xv = jnp.concatenate([x_ref[0, 0] + x_ref[0, 1],
                                  x_ref[1, 0] + x_ref[1, 1]], axis=1)
        elif split_in:
            xv = jnp.concatenate([x_ref[0], x_ref[1]], axis=1)
        else:
            xv = x_ref[...]
        h = _ssp(jnp.dot(xv, w1_ref[...],
                         preferred_element_type=jnp.float32) + b1_ref[...])
        o = (jnp.dot(h, w2_ref[...], preferred_element_type=jnp.float32)
             + b2_ref[...])
        if res is not None:
            o = o + r_ref[...]
        if split_out:
            o_ref[0] = o[:, :_HW]
            o_ref[1] = o[:, _HW:]
        else:
            o_ref[...] = o

    if drp_in:
        x_spec = pl.BlockSpec((2, 2, _BLK, _HW), lambda i: (0, 0, i, 0))
    elif split_in:
        x_spec = pl.BlockSpec((2, _BLK, _HW), lambda i: (0, i, 0))
    else:
        x_spec = _row_spec(_BLK, x.shape[1])

    in_specs = [
        x_spec,
        _full_spec(w1.shape),
        _full_spec(b1.shape),
        _full_spec(w2.shape),
        _full_spec(b2.shape),
    ]
    args = [x, w1, b1, w2, b2]
    if res is not None:
        in_specs.append(_row_spec(_BLK, dout))
        args.append(res)

    if split_out:
        out_spec = pl.BlockSpec((2, _BLK, _HW), lambda i: (0, i, 0))
        out_shape = jax.ShapeDtypeStruct((2, rows, _HW), jnp.float32)
    else:
        out_spec = _row_spec(_BLK, dout)
        out_shape = jax.ShapeDtypeStruct((rows, dout), jnp.float32)

    return pl.pallas_call(
        body,
        grid=(grid,),
        in_specs=in_specs,
        out_specs=out_spec,
        out_shape=out_shape,
        compiler_params=_tc_params(),
    )(*args)


def _tc_readout(e, adj, p1, p2):
    rows, dcoord = adj.shape
    grid = rows // _BLK
    w1, b1 = p1[0], p1[1].reshape(1, -1)
    w2, b2 = p2[0], p2[1].reshape(1, -1)

    def body(e_ref, a_ref, w1_ref, b1_ref, w2_ref, b2_ref, fp_ref, fn_ref):
        h = _ssp(jnp.dot(e_ref[...], w1_ref[...],
                         preferred_element_type=jnp.float32) + b1_ref[...])
        val = (jnp.dot(h, w2_ref[...], preferred_element_type=jnp.float32)
               + b2_ref[...])
        fp = val * a_ref[...]
        fp_ref[...] = fp
        fn_ref[...] = -fp

    return pl.pallas_call(
        body,
        grid=(grid,),
        in_specs=[
            _row_spec(_BLK, 128),
            _row_spec(_BLK, dcoord),
            _full_spec(w1.shape),
            _full_spec(b1.shape),
            _full_spec(w2.shape),
            _full_spec(b2.shape),
        ],
        out_specs=[_row_spec(_BLK, dcoord), _row_spec(_BLK, dcoord)],
        out_shape=[
            jax.ShapeDtypeStruct((rows, dcoord), jnp.float32),
            jax.ShapeDtypeStruct((rows, dcoord), jnp.float32),
        ],
        compiler_params=_tc_params(),
    )(e, adj, w1, b1, w2, b2)


def _tc_combine(facc):
    rows, d = facc.shape[1], facc.shape[2]
    grid = rows // _BLK

    def body(f_ref, o_ref):
        o_ref[...] = f_ref[0] + f_ref[1]

    return pl.pallas_call(
        body,
        grid=(grid,),
        in_specs=[pl.BlockSpec((2, _BLK, d), lambda i: (0, i, 0))],
        out_specs=_row_spec(_BLK, d),
        out_shape=jax.ShapeDtypeStruct((rows, d), jnp.float32),
        compiler_params=_tc_params(),
    )(facc)


def kernel(nxyz, nbr_list, params):
    n = nxyz.shape[0]
    e_cnt = nbr_list.shape[0]
    npad = -(-(n + 1) // 2048) * 2048
    nblk = -(-e_cnt // (NW * 128))
    epad = NW * 128 * nblk

    z = nxyz[:, 0].astype(jnp.int32)
    xyz = nxyz[:, 1:4].astype(jnp.float32)
    xyz_pad = jnp.zeros((npad, 16), jnp.float32).at[:n, :3].set(xyz)

    a = nbr_list.astype(jnp.int32)
    a0 = jnp.full((epad,), n, jnp.int32).at[:e_cnt].set(a[:, 0])
    a1 = jnp.full((epad,), n, jnp.int32).at[:e_cnt].set(a[:, 1])
    z_pad = jnp.zeros((npad,), jnp.int32).at[:n].set(z)

    emb = params['emb']
    emb_pad = jnp.zeros((128, emb.shape[1]), jnp.float32).at[:emb.shape[0]].set(emb)

    r = _sc_gather(emb_pad, z_pad, 64)
    dvec = _sc_dvec(xyz_pad, a0, a1)

    offs = jnp.linspace(0.0, _CUTOFF, _NG).astype(jnp.float32)
    width = float(_CUTOFF / (_NG - 1))
    offs = jnp.concatenate([offs, jnp.full((14,), 1e9, jnp.float32)])
    offs = offs.reshape(1, 64)

    efp = params['ef']
    w1 = jnp.zeros((64, efp[0][0].shape[1]), jnp.float32).at[:_NG].set(efp[0][0])
    e, adj = _tc_geom(dvec, w1, efp[0][1].reshape(1, -1),
                      efp[1][0], efp[1][1].reshape(1, -1), offs, width)

    for cp in params['convs']:
        ef2 = _tc_mlp(e, cp['edge_filter'][0], cp['edge_filter'][1],
                      split_out=True)
        rf2 = _tc_mlp(r, cp['atom_filter'][0], cp['atom_filter'][1],
                      split_out=True)
        s2, drp = _sc_convmsg(rf2, ef2, a0, a1, npad)
        r = _tc_mlp(drp, cp['atom_update'][0], cp['atom_update'][1],
                    res=r, drp_in=True)
        e = _tc_mlp(s2, cp['edge_update'][0], cp['edge_update'][1],
                    res=e, split_in=True)

    fpos, fneg = _tc_readout(e, adj, params['ro'][0], params['ro'][1])
    facc = _sc_fscatter(fpos, fneg, a0, a1, npad)
    f_atom = _tc_combine(facc)
    return f_atom[:n, :3]


# speedup vs baseline: 1.3764x; 1.2066x over previous
import functools

import jax
import jax.numpy as jnp
from jax import lax
from jax.experimental import pallas as pl
from jax.experimental.pallas import tpu as pltpu
from jax.experimental.pallas import tpu_sc as plsc

NC, NS, LANES = 2, 16, 16
NW = NC * NS

_LN2 = 0.6931471805599453
_CUTOFF = 5.0
_NG = 50
_HW = 64


def _ssp(x):
    return jax.nn.softplus(x) - _LN2


def _sc_mesh():
    return plsc.VectorSubcoreMesh(core_axis_name="c", subcore_axis_name="s")


_SC_CP = pltpu.CompilerParams(use_tc_tiling_on_sc=False)


def _sc_gather(tbl, idx, bsz):
    rows = idx.shape[0]
    d = tbl.shape[1]
    nblk = rows // (NW * bsz)

    @functools.partial(
        pl.kernel,
        out_type=pltpu.HBM((rows, d), jnp.float32),
        mesh=_sc_mesh(),
        scratch_types=[
            pltpu.VMEM((bsz,), jnp.int32),
            pltpu.VMEM((bsz, d), jnp.float32),
            pltpu.SemaphoreType.DMA,
        ],
    )
    def k(tbl_ref, idx_ref, out_ref, idx_v, rows_v, sem):
        wid = lax.axis_index("s") * NC + lax.axis_index("c")

        def body(i, c):
            base = (wid * nblk + i) * bsz
            pltpu.sync_copy(idx_ref.at[pl.ds(base, bsz)], idx_v)
            pltpu.async_copy(tbl_ref.at[idx_v], rows_v, sem).wait()
            pltpu.sync_copy(rows_v, out_ref.at[pl.ds(base, bsz)])
            return c

        lax.fori_loop(0, nblk, body, 0)

    return k(tbl, idx)


def _sc_dvec(xyz_pad, a0, a1):
    rows = a0.shape[0]
    d = xyz_pad.shape[1]
    bsz = 128
    nblk = rows // (NW * bsz)

    @functools.partial(
        pl.kernel,
        out_type=pltpu.HBM((rows, d), jnp.float32),
        mesh=_sc_mesh(),
        compiler_params=_SC_CP,
        scratch_types=[
            pltpu.VMEM((bsz,), jnp.int32),
            pltpu.VMEM((bsz,), jnp.int32),
            pltpu.VMEM((bsz, d), jnp.float32),
            pltpu.VMEM((bsz, d), jnp.float32),
            pltpu.SemaphoreType.DMA,
            pltpu.SemaphoreType.DMA,
        ],
    )
    def k(tbl_ref, a0_ref, a1_ref, out_ref, i0_v, i1_v, r0_v, r1_v, s0, s1):
        wid = lax.axis_index("s") * NC + lax.axis_index("c")

        def body(i, c):
            base = (wid * nblk + i) * bsz
            pltpu.sync_copy(a0_ref.at[pl.ds(base, bsz)], i0_v)
            pltpu.sync_copy(a1_ref.at[pl.ds(base, bsz)], i1_v)
            c0 = pltpu.async_copy(tbl_ref.at[i0_v], r0_v, s0)
            c1 = pltpu.async_copy(tbl_ref.at[i1_v], r1_v, s1)
            c0.wait()
            c1.wait()

            def row(j, cc):
                r0_v[j] = r0_v[j] - r1_v[j]
                return cc

            lax.fori_loop(0, bsz, row, 0)
            pltpu.sync_copy(r0_v, out_ref.at[pl.ds(base, bsz)])
            return c

        lax.fori_loop(0, nblk, body, 0)

    return k(xyz_pad, a0, a1)


def _sc_convmsg(rf, ef, a0, a1, npad):
    rows = a0.shape[0]
    bsz = 64
    d = rf.shape[1]
    nblk = rows // (NW * bsz)
    rps = npad // NS
    nchunk = rps // bsz

    @functools.partial(
        pl.kernel,
        out_type=(
            pltpu.HBM((rows, d), jnp.float32),
            pltpu.HBM((2, npad, d), jnp.float32),
        ),
        mesh=_sc_mesh(),
        compiler_params=_SC_CP,
        scratch_types=[
            pltpu.VMEM((bsz,), jnp.int32),
            pltpu.VMEM((bsz,), jnp.int32),
            pltpu.VMEM((bsz, d), jnp.float32),
            pltpu.VMEM((bsz, d), jnp.float32),
            pltpu.VMEM((bsz, d), jnp.float32),
            pltpu.VMEM((bsz, d), jnp.float32),
            pltpu.VMEM_SHARED((npad, d), jnp.float32),
            pltpu.SemaphoreType.DMA,
            pltpu.SemaphoreType.DMA,
        ],
    )
    def k(rf_ref, ef_ref, a0_ref, a1_ref, s_out, drp_out,
          i0_v, i1_v, ef_v, r0_v, r1_v, s_v, acc, sem0, sem1):
        cid = lax.axis_index("c")
        sid = lax.axis_index("s")
        wid = sid * NC + cid

        def zero_sv(j, c):
            for kk in range(d // 16):
                s_v[j, pl.ds(kk * 16, 16)] = jnp.zeros((16,), jnp.float32)
            return c

        lax.fori_loop(0, bsz, zero_sv, 0)

        def zc(q, c):
            pltpu.sync_copy(s_v, acc.at[pl.ds(sid * rps + q * bsz, bsz)])
            return c

        lax.fori_loop(0, nchunk, zc, 0)
        plsc.subcore_barrier()

        def body(i, c):
            base = (wid * nblk + i) * bsz
            pltpu.sync_copy(a0_ref.at[pl.ds(base, bsz)], i0_v)
            pltpu.sync_copy(a1_ref.at[pl.ds(base, bsz)], i1_v)
            c0 = pltpu.async_copy(rf_ref.at[i0_v], r0_v, sem0)
            c1 = pltpu.async_copy(rf_ref.at[i1_v], r1_v, sem1)
            pltpu.sync_copy(ef_ref.at[pl.ds(base, bsz)], ef_v)
            c0.wait()
            c1.wait()

            def row(j, cc):
                for kk in range(d // 16):
                    dsl = pl.ds(kk * 16, 16)
                    e = ef_v[j, dsl]
                    x0 = r0_v[j, dsl] * e
                    x1 = r1_v[j, dsl] * e
                    r0_v[j, dsl] = x0
                    r1_v[j, dsl] = x1
                    s_v[j, dsl] = x0 + x1
                return cc

            lax.fori_loop(0, bsz, row, 0)
            pltpu.sync_copy(s_v, s_out.at[pl.ds(base, bsz)])
            pltpu.sync_copy(r0_v, acc.at[i1_v], add=True)
            pltpu.sync_copy(r1_v, acc.at[i0_v], add=True)
            return c

        lax.fori_loop(0, nblk, body, 0)
        plsc.subcore_barrier()

        def dump(q, c):
            ro = sid * rps + q * bsz
            pltpu.sync_copy(acc.at[pl.ds(ro, bsz)], s_v)
            pltpu.sync_copy(s_v, drp_out.at[cid, pl.ds(ro, bsz)])
            return c

        lax.fori_loop(0, nchunk, dump, 0)

    return k(rf, ef, a0, a1)


def _sc_fscatter(fpos, fneg, a0, a1, npad):
    rows = a0.shape[0]
    d = fpos.shape[1]
    bsz = 128
    nblk = rows // (NW * bsz)
    rps = npad // NS
    nchunk = rps // bsz

    @functools.partial(
        pl.kernel,
        out_type=pltpu.HBM((2, npad, d), jnp.float32),
        mesh=_sc_mesh(),
        compiler_params=_SC_CP,
        scratch_types=[
            pltpu.VMEM((bsz,), jnp.int32),
            pltpu.VMEM((bsz,), jnp.int32),
            pltpu.VMEM((bsz, d), jnp.float32),
            pltpu.VMEM((bsz, d), jnp.float32),
            pltpu.VMEM_SHARED((npad, d), jnp.float32),
        ],
    )
    def k(fp_ref, fn_ref, a0_ref, a1_ref, out_ref,
          i0_v, i1_v, f0_v, f1_v, acc):
        cid = lax.axis_index("c")
        sid = lax.axis_index("s")
        wid = sid * NC + cid

        def zrow(j, c):
            f0_v[j] = jnp.zeros((d,), jnp.float32)
            return c

        lax.fori_loop(0, bsz, zrow, 0)

        def zc(q, c):
            pltpu.sync_copy(f0_v, acc.at[pl.ds(sid * rps + q * bsz, bsz)])
            return c

        lax.fori_loop(0, nchunk, zc, 0)
        plsc.subcore_barrier()

        def body(i, c):
            base = (wid * nblk + i) * bsz
            pltpu.sync_copy(a0_ref.at[pl.ds(base, bsz)], i0_v)
            pltpu.sync_copy(a1_ref.at[pl.ds(base, bsz)], i1_v)
            pltpu.sync_copy(fp_ref.at[pl.ds(base, bsz)], f0_v)
            pltpu.sync_copy(fn_ref.at[pl.ds(base, bsz)], f1_v)
            pltpu.sync_copy(f0_v, acc.at[i0_v], add=True)
            pltpu.sync_copy(f1_v, acc.at[i1_v], add=True)
            return c

        lax.fori_loop(0, nblk, body, 0)
        plsc.subcore_barrier()

        def dump(q, c):
            ro = sid * rps + q * bsz
            pltpu.sync_copy(acc.at[pl.ds(ro, bsz)], f0_v)
            pltpu.sync_copy(f0_v, out_ref.at[cid, pl.ds(ro, bsz)])
            return c

        lax.fori_loop(0, nchunk, dump, 0)

    return k(fpos, fneg, a0, a1)


_BLK = 512


def _row_spec(blk, d):
    return pl.BlockSpec((blk, d), lambda i: (i, 0))


def _full_spec(shape):
    nd = len(shape)
    return pl.BlockSpec(shape, lambda i: (0,) * nd)


def _tc_params():
    return pltpu.CompilerParams(dimension_semantics=("parallel",))


def _tc_geom(dvec, w1, b1, w2, b2, offs, width):
    rows, dcoord = dvec.shape
    grid = rows // _BLK

    def body(dv_ref, offs_ref, w1_ref, b1_ref, w2_ref, b2_ref, e_ref, adj_ref):
        dv = dv_ref[...]
        d2 = jnp.sum(dv * dv, axis=1, keepdims=True)
        dis = jnp.sqrt(d2)
        adj_ref[...] = dv / dis
        x = (dis - offs_ref[...]) / width
        e0 = jnp.exp(-0.5 * x * x)
        h = _ssp(jnp.dot(e0, w1_ref[...], preferred_element_type=jnp.float32)
                 + b1_ref[...])
        e_ref[...] = (jnp.dot(h, w2_ref[...], preferred_element_type=jnp.float32)
                      + b2_ref[...])

    return pl.pallas_call(
        body,
        grid=(grid,),
        in_specs=[
            _row_spec(_BLK, dcoord),
            _full_spec(offs.shape),
            _full_spec(w1.shape),
            _full_spec(b1.shape),
            _full_spec(w2.shape),
            _full_spec(b2.shape),
        ],
        out_specs=[_row_spec(_BLK, 128), _row_spec(_BLK, dcoord)],
        out_shape=[
            jax.ShapeDtypeStruct((rows, 128), jnp.float32),
            jax.ShapeDtypeStruct((rows, dcoord), jnp.float32),
        ],
        compiler_params=_tc_params(),
    )(dvec, offs, w1, b1, w2, b2)


def _tc_mlp(x, p1, p2, res=None, drp_in=False):
    rows = x.shape[1] if drp_in else x.shape[0]
    grid = rows // _BLK
    w1, b1 = p1[0], p1[1].reshape(1, -1)
    w2, b2 = p2[0], p2[1].reshape(1, -1)
    dout = w2.shape[1]

    def body(*refs):
        refs = list(refs)
        x_ref = refs.pop(0)
        w1_ref, b1_ref, w2_ref, b2_ref = refs[:4]
        refs = refs[4:]
        r_ref = refs.pop(0) if res is not None else None
        o_ref = refs.pop(0)
        if drp_in:
            xv = x_ref[0] + x_ref[1]
        else:
            xv = x_ref[...]
        h = _ssp(jnp.dot(xv, w1_ref[...],
                         preferred_element_type=jnp.float32) + b1_ref[...])
        o = (jnp.dot(h, w2_ref[...], preferred_element_type=jnp.float32)
             + b2_ref[...])
        if res is not None:
            o = o + r_ref[...]
        o_ref[...] = o

    if drp_in:
        x_spec = pl.BlockSpec((2, _BLK, x.shape[2]), lambda i: (0, i, 0))
    else:
        x_spec = _row_spec(_BLK, x.shape[1])

    in_specs = [
        x_spec,
        _full_spec(w1.shape),
        _full_spec(b1.shape),
        _full_spec(w2.shape),
        _full_spec(b2.shape),
    ]
    args = [x, w1, b1, w2, b2]
    if res is not None:
        in_specs.append(_row_spec(_BLK, dout))
        args.append(res)

    return pl.pallas_call(
        body,
        grid=(grid,),
        in_specs=in_specs,
        out_specs=_row_spec(_BLK, dout),
        out_shape=jax.ShapeDtypeStruct((rows, dout), jnp.float32),
        compiler_params=_tc_params(),
    )(*args)


def _tc_readout(e, adj, p1, p2):
    rows, dcoord = adj.shape
    grid = rows // _BLK
    w1, b1 = p1[0], p1[1].reshape(1, -1)
    w2, b2 = p2[0], p2[1].reshape(1, -1)

    def body(e_ref, a_ref, w1_ref, b1_ref, w2_ref, b2_ref, fp_ref, fn_ref):
        h = _ssp(jnp.dot(e_ref[...], w1_ref[...],
                         preferred_element_type=jnp.float32) + b1_ref[...])
        val = (jnp.dot(h, w2_ref[...], preferred_element_type=jnp.float32)
               + b2_ref[...])
        fp = val * a_ref[...]
        fp_ref[...] = fp
        fn_ref[...] = -fp

    return pl.pallas_call(
        body,
        grid=(grid,),
        in_specs=[
            _row_spec(_BLK, 128),
            _row_spec(_BLK, dcoord),
            _full_spec(w1.shape),
            _full_spec(b1.shape),
            _full_spec(w2.shape),
            _full_spec(b2.shape),
        ],
        out_specs=[_row_spec(_BLK, dcoord), _row_spec(_BLK, dcoord)],
        out_shape=[
            jax.ShapeDtypeStruct((rows, dcoord), jnp.float32),
            jax.ShapeDtypeStruct((rows, dcoord), jnp.float32),
        ],
        compiler_params=_tc_params(),
    )(e, adj, w1, b1, w2, b2)


def _tc_combine(facc):
    rows, d = facc.shape[1], facc.shape[2]
    grid = rows // _BLK

    def body(f_ref, o_ref):
        o_ref[...] = f_ref[0] + f_ref[1]

    return pl.pallas_call(
        body,
        grid=(grid,),
        in_specs=[pl.BlockSpec((2, _BLK, d), lambda i: (0, i, 0))],
        out_specs=_row_spec(_BLK, d),
        out_shape=jax.ShapeDtypeStruct((rows, d), jnp.float32),
        compiler_params=_tc_params(),
    )(facc)


def kernel(nxyz, nbr_list, params):
    n = nxyz.shape[0]
    e_cnt = nbr_list.shape[0]
    npad = -(-(n + 1) // 2048) * 2048
    nblk = -(-e_cnt // (NW * 128))
    epad = NW * 128 * nblk

    z = nxyz[:, 0].astype(jnp.int32)
    xyz = nxyz[:, 1:4].astype(jnp.float32)
    xyz_pad = jnp.zeros((npad, 16), jnp.float32).at[:n, :3].set(xyz)

    a = nbr_list.astype(jnp.int32)
    a0 = jnp.full((epad,), n, jnp.int32).at[:e_cnt].set(a[:, 0])
    a1 = jnp.full((epad,), n, jnp.int32).at[:e_cnt].set(a[:, 1])
    z_pad = jnp.zeros((npad,), jnp.int32).at[:n].set(z)

    emb = params['emb']
    emb_pad = jnp.zeros((128, emb.shape[1]), jnp.float32).at[:emb.shape[0]].set(emb)

    r = _sc_gather(emb_pad, z_pad, 64)
    dvec = _sc_dvec(xyz_pad, a0, a1)

    offs = jnp.linspace(0.0, _CUTOFF, _NG).astype(jnp.float32)
    width = float(_CUTOFF / (_NG - 1))
    offs = jnp.concatenate([offs, jnp.full((14,), 1e9, jnp.float32)])
    offs = offs.reshape(1, 64)

    efp = params['ef']
    w1 = jnp.zeros((64, efp[0][0].shape[1]), jnp.float32).at[:_NG].set(efp[0][0])
    e, adj = _tc_geom(dvec, w1, efp[0][1].reshape(1, -1),
                      efp[1][0], efp[1][1].reshape(1, -1), offs, width)

    for cp in params['convs']:
        ef = _tc_mlp(e, cp['edge_filter'][0], cp['edge_filter'][1])
        rf = _tc_mlp(r, cp['atom_filter'][0], cp['atom_filter'][1])
        s, drp = _sc_convmsg(rf, ef, a0, a1, npad)
        r = _tc_mlp(drp, cp['atom_update'][0], cp['atom_update'][1],
                    res=r, drp_in=True)
        e = _tc_mlp(s, cp['edge_update'][0], cp['edge_update'][1],
                    res=e)

    fpos, fneg = _tc_readout(e, adj, params['ro'][0], params['ro'][1])
    facc = _sc_fscatter(fpos, fneg, a0, a1, npad)
    f_atom = _tc_combine(facc)
    return f_atom[:n, :3]


# speedup vs baseline: 1.5117x; 1.0983x over previous
import functools

import jax
import jax.numpy as jnp
from jax import lax
from jax.experimental import pallas as pl
from jax.experimental.pallas import tpu as pltpu
from jax.experimental.pallas import tpu_sc as plsc

NC, NS, LANES = 2, 16, 16
NW = NC * NS

_LN2 = 0.6931471805599453
_CUTOFF = 5.0
_NG = 50
_HW = 64


def _ssp(x):
    return jax.nn.softplus(x) - _LN2


def _sc_mesh():
    return plsc.VectorSubcoreMesh(core_axis_name="c", subcore_axis_name="s")


_SC_CP = pltpu.CompilerParams(use_tc_tiling_on_sc=False)


def _sc_gather(tbl, idx, bsz):
    rows = idx.shape[0]
    d = tbl.shape[1]
    nblk = rows // (NW * bsz)

    @functools.partial(
        pl.kernel,
        out_type=pltpu.HBM((rows, d), jnp.float32),
        mesh=_sc_mesh(),
        scratch_types=[
            pltpu.VMEM((bsz,), jnp.int32),
            pltpu.VMEM((bsz, d), jnp.float32),
            pltpu.SemaphoreType.DMA,
        ],
    )
    def k(tbl_ref, idx_ref, out_ref, idx_v, rows_v, sem):
        wid = lax.axis_index("s") * NC + lax.axis_index("c")

        def body(i, c):
            base = (wid * nblk + i) * bsz
            pltpu.sync_copy(idx_ref.at[pl.ds(base, bsz)], idx_v)
            pltpu.async_copy(tbl_ref.at[idx_v], rows_v, sem).wait()
            pltpu.sync_copy(rows_v, out_ref.at[pl.ds(base, bsz)])
            return c

        lax.fori_loop(0, nblk, body, 0)

    return k(tbl, idx)


def _sc_dvec(xyz_pad, a0, a1):
    rows = a0.shape[0]
    d = xyz_pad.shape[1]
    bsz = 128
    nblk = rows // (NW * bsz)

    @functools.partial(
        pl.kernel,
        out_type=pltpu.HBM((rows, d), jnp.float32),
        mesh=_sc_mesh(),
        compiler_params=_SC_CP,
        scratch_types=[
            pltpu.VMEM((bsz,), jnp.int32),
            pltpu.VMEM((bsz,), jnp.int32),
            pltpu.VMEM((bsz, d), jnp.float32),
            pltpu.VMEM((bsz, d), jnp.float32),
            pltpu.SemaphoreType.DMA,
            pltpu.SemaphoreType.DMA,
        ],
    )
    def k(tbl_ref, a0_ref, a1_ref, out_ref, i0_v, i1_v, r0_v, r1_v, s0, s1):
        wid = lax.axis_index("s") * NC + lax.axis_index("c")

        def body(i, c):
            base = (wid * nblk + i) * bsz
            pltpu.sync_copy(a0_ref.at[pl.ds(base, bsz)], i0_v)
            pltpu.sync_copy(a1_ref.at[pl.ds(base, bsz)], i1_v)
            c0 = pltpu.async_copy(tbl_ref.at[i0_v], r0_v, s0)
            c1 = pltpu.async_copy(tbl_ref.at[i1_v], r1_v, s1)
            c0.wait()
            c1.wait()

            def row(j, cc):
                r0_v[j] = r0_v[j] - r1_v[j]
                return cc

            lax.fori_loop(0, bsz, row, 0)
            pltpu.sync_copy(r0_v, out_ref.at[pl.ds(base, bsz)])
            return c

        lax.fori_loop(0, nblk, body, 0)

    return k(xyz_pad, a0, a1)


def _sc_convmsg(rf, ef, a0, a1, npad):
    rows = a0.shape[0]
    bsz = 64
    d = rf.shape[1]
    nblk = rows // (NW * bsz)
    rps = npad // NS
    nchunk = rps // bsz

    @functools.partial(
        pl.kernel,
        out_type=(
            pltpu.HBM((rows, d), jnp.float32),
            pltpu.HBM((2, npad, d), jnp.float32),
        ),
        mesh=_sc_mesh(),
        compiler_params=_SC_CP,
        scratch_types=[
            pltpu.VMEM((bsz,), jnp.int32),
            pltpu.VMEM((bsz,), jnp.int32),
            pltpu.VMEM((bsz, d), jnp.float32),
            pltpu.VMEM((bsz, d), jnp.float32),
            pltpu.VMEM((bsz, d), jnp.float32),
            pltpu.VMEM((bsz, d), jnp.float32),
            pltpu.VMEM_SHARED((npad, d), jnp.float32),
            pltpu.SemaphoreType.DMA,
            pltpu.SemaphoreType.DMA,
        ],
    )
    def k(rf_ref, ef_ref, a0_ref, a1_ref, s_out, drp_out,
          i0_v, i1_v, ef_v, r0_v, r1_v, s_v, acc, sem0, sem1):
        cid = lax.axis_index("c")
        sid = lax.axis_index("s")
        wid = sid * NC + cid

        def zero_sv(j, c):
            for kk in range(d // 16):
                s_v[j, pl.ds(kk * 16, 16)] = jnp.zeros((16,), jnp.float32)
            return c

        lax.fori_loop(0, bsz, zero_sv, 0)

        def zc(q, c):
            pltpu.sync_copy(s_v, acc.at[pl.ds(sid * rps + q * bsz, bsz)])
            return c

        lax.fori_loop(0, nchunk, zc, 0)
        plsc.subcore_barrier()

        def body(i, c):
            base = (wid * nblk + i) * bsz
            pltpu.sync_copy(a0_ref.at[pl.ds(base, bsz)], i0_v)
            pltpu.sync_copy(a1_ref.at[pl.ds(base, bsz)], i1_v)
            c0 = pltpu.async_copy(rf_ref.at[i0_v], r0_v, sem0)
            c1 = pltpu.async_copy(rf_ref.at[i1_v], r1_v, sem1)
            pltpu.sync_copy(ef_ref.at[pl.ds(base, bsz)], ef_v)
            c0.wait()
            c1.wait()

            def row(j, cc):
                for kk in range(d // 16):
                    dsl = pl.ds(kk * 16, 16)
                    e = ef_v[j, dsl]
                    x0 = r0_v[j, dsl] * e
                    x1 = r1_v[j, dsl] * e
                    r0_v[j, dsl] = x0
                    r1_v[j, dsl] = x1
                    s_v[j, dsl] = x0 + x1
                return cc

            lax.fori_loop(0, bsz, row, 0)
            pltpu.sync_copy(s_v, s_out.at[pl.ds(base, bsz)])
            pltpu.sync_copy(r0_v, acc.at[i1_v], add=True)
            pltpu.sync_copy(r1_v, acc.at[i0_v], add=True)
            return c

        lax.fori_loop(0, nblk, body, 0)
        plsc.subcore_barrier()

        def dump(q, c):
            ro = sid * rps + q * bsz
            pltpu.sync_copy(acc.at[pl.ds(ro, bsz)], s_v)
            pltpu.sync_copy(s_v, drp_out.at[cid, pl.ds(ro, bsz)])
            return c

        lax.fori_loop(0, nchunk, dump, 0)

    return k(rf, ef, a0, a1)


def _sc_fscatter(fpos, fneg, a0, a1, npad):
    rows = a0.shape[0]
    d = fpos.shape[1]
    bsz = 128
    nblk = rows // (NW * bsz)
    rps = npad // NS
    nchunk = rps // bsz

    @functools.partial(
        pl.kernel,
        out_type=pltpu.HBM((2, npad, d), jnp.float32),
        mesh=_sc_mesh(),
        compiler_params=_SC_CP,
        scratch_types=[
            pltpu.VMEM((bsz,), jnp.int32),
            pltpu.VMEM((bsz,), jnp.int32),
            pltpu.VMEM((bsz, d), jnp.float32),
            pltpu.VMEM((bsz, d), jnp.float32),
            pltpu.VMEM_SHARED((npad, d), jnp.float32),
        ],
    )
    def k(fp_ref, fn_ref, a0_ref, a1_ref, out_ref,
          i0_v, i1_v, f0_v, f1_v, acc):
        cid = lax.axis_index("c")
        sid = lax.axis_index("s")
        wid = sid * NC + cid

        def zrow(j, c):
            f0_v[j] = jnp.zeros((d,), jnp.float32)
            return c

        lax.fori_loop(0, bsz, zrow, 0)

        def zc(q, c):
            pltpu.sync_copy(f0_v, acc.at[pl.ds(sid * rps + q * bsz, bsz)])
            return c

        lax.fori_loop(0, nchunk, zc, 0)
        plsc.subcore_barrier()

        def body(i, c):
            base = (wid * nblk + i) * bsz
            pltpu.sync_copy(a0_ref.at[pl.ds(base, bsz)], i0_v)
            pltpu.sync_copy(a1_ref.at[pl.ds(base, bsz)], i1_v)
            pltpu.sync_copy(fp_ref.at[pl.ds(base, bsz)], f0_v)
            pltpu.sync_copy(fn_ref.at[pl.ds(base, bsz)], f1_v)
            pltpu.sync_copy(f0_v, acc.at[i0_v], add=True)
            pltpu.sync_copy(f1_v, acc.at[i1_v], add=True)
            return c

        lax.fori_loop(0, nblk, body, 0)
        plsc.subcore_barrier()

        def dump(q, c):
            ro = sid * rps + q * bsz
            pltpu.sync_copy(acc.at[pl.ds(ro, bsz)], f0_v)
            pltpu.sync_copy(f0_v, out_ref.at[cid, pl.ds(ro, bsz)])
            return c

        lax.fori_loop(0, nchunk, dump, 0)

    return k(fpos, fneg, a0, a1)


_BLK = 512


def _row_spec(blk, d):
    return pl.BlockSpec((blk, d), lambda i: (i, 0))


def _full_spec(shape):
    nd = len(shape)
    return pl.BlockSpec(shape, lambda i: (0,) * nd)


def _tc_params():
    return pltpu.CompilerParams(dimension_semantics=("parallel",))


def _tc_geom(dvec, w1, b1, w2, b2, offs, width):
    rows, dcoord = dvec.shape
    grid = rows // _BLK

    def body(dv_ref, offs_ref, w1_ref, b1_ref, w2_ref, b2_ref, e_ref, adj_ref):
        dv = dv_ref[...]
        d2 = jnp.sum(dv * dv, axis=1, keepdims=True)
        dis = jnp.sqrt(d2)
        adj_ref[...] = dv / dis
        x = (dis - offs_ref[...]) / width
        e0 = jnp.exp(-0.5 * x * x)
        h = _ssp(jnp.dot(e0, w1_ref[...], preferred_element_type=jnp.float32)
                 + b1_ref[...])
        e_ref[...] = (jnp.dot(h, w2_ref[...], preferred_element_type=jnp.float32)
                      + b2_ref[...])

    return pl.pallas_call(
        body,
        grid=(grid,),
        in_specs=[
            _row_spec(_BLK, dcoord),
            _full_spec(offs.shape),
            _full_spec(w1.shape),
            _full_spec(b1.shape),
            _full_spec(w2.shape),
            _full_spec(b2.shape),
        ],
        out_specs=[_row_spec(_BLK, 128), _row_spec(_BLK, dcoord)],
        out_shape=[
            jax.ShapeDtypeStruct((rows, 128), jnp.float32),
            jax.ShapeDtypeStruct((rows, dcoord), jnp.float32),
        ],
        compiler_params=_tc_params(),
    )(dvec, offs, w1, b1, w2, b2)


def _tc_mlp(x, p1, p2, res=None, drp_in=False):
    rows = x.shape[1] if drp_in else x.shape[0]
    grid = rows // _BLK
    w1, b1 = p1[0], p1[1].reshape(1, -1)
    w2, b2 = p2[0], p2[1].reshape(1, -1)
    dout = w2.shape[1]

    nsum = x.shape[0] if drp_in else 0

    def body(*refs):
        refs = list(refs)
        x_ref = refs.pop(0)
        w1_ref, b1_ref, w2_ref, b2_ref = refs[:4]
        refs = refs[4:]
        r_ref = refs.pop(0) if res is not None else None
        o_ref = refs.pop(0)
        if drp_in:
            xv = x_ref[0]
            for q in range(1, nsum):
                xv = xv + x_ref[q]
        else:
            xv = x_ref[...]
        h = _ssp(jnp.dot(xv, w1_ref[...],
                         preferred_element_type=jnp.float32) + b1_ref[...])
        o = (jnp.dot(h, w2_ref[...], preferred_element_type=jnp.float32)
             + b2_ref[...])
        if res is not None:
            o = o + r_ref[...]
        o_ref[...] = o

    if drp_in:
        x_spec = pl.BlockSpec((nsum, _BLK, x.shape[2]),
                              lambda i: (0, i, 0))
    else:
        x_spec = _row_spec(_BLK, x.shape[1])

    in_specs = [
        x_spec,
        _full_spec(w1.shape),
        _full_spec(b1.shape),
        _full_spec(w2.shape),
        _full_spec(b2.shape),
    ]
    args = [x, w1, b1, w2, b2]
    if res is not None:
        in_specs.append(_row_spec(_BLK, dout))
        args.append(res)

    return pl.pallas_call(
        body,
        grid=(grid,),
        in_specs=in_specs,
        out_specs=_row_spec(_BLK, dout),
        out_shape=jax.ShapeDtypeStruct((rows, dout), jnp.float32),
        compiler_params=_tc_params(),
    )(*args)


def _tc_readout(e, adj, p1, p2):
    rows, dcoord = adj.shape
    grid = rows // _BLK
    w1, b1 = p1[0], p1[1].reshape(1, -1)
    w2, b2 = p2[0], p2[1].reshape(1, -1)

    def body(e_ref, a_ref, w1_ref, b1_ref, w2_ref, b2_ref, fp_ref, fn_ref):
        h = _ssp(jnp.dot(e_ref[...], w1_ref[...],
                         preferred_element_type=jnp.float32) + b1_ref[...])
        val = (jnp.dot(h, w2_ref[...], preferred_element_type=jnp.float32)
               + b2_ref[...])
        fp = val * a_ref[...]
        fp_ref[...] = fp
        fn_ref[...] = -fp

    return pl.pallas_call(
        body,
        grid=(grid,),
        in_specs=[
            _row_spec(_BLK, 128),
            _row_spec(_BLK, dcoord),
            _full_spec(w1.shape),
            _full_spec(b1.shape),
            _full_spec(w2.shape),
            _full_spec(b2.shape),
        ],
        out_specs=[_row_spec(_BLK, dcoord), _row_spec(_BLK, dcoord)],
        out_shape=[
            jax.ShapeDtypeStruct((rows, dcoord), jnp.float32),
            jax.ShapeDtypeStruct((rows, dcoord), jnp.float32),
        ],
        compiler_params=_tc_params(),
    )(e, adj, w1, b1, w2, b2)


def _tc_combine(facc):
    k, rows, d = facc.shape
    grid = rows // _BLK

    def body(f_ref, o_ref):
        o = f_ref[0]
        for q in range(1, k):
            o = o + f_ref[q]
        o_ref[...] = o

    return pl.pallas_call(
        body,
        grid=(grid,),
        in_specs=[pl.BlockSpec((k, _BLK, d), lambda i: (0, i, 0))],
        out_specs=_row_spec(_BLK, d),
        out_shape=jax.ShapeDtypeStruct((rows, d), jnp.float32),
        compiler_params=_tc_params(),
    )(facc)


def kernel(nxyz, nbr_list, params):
    n = nxyz.shape[0]
    e_cnt = nbr_list.shape[0]
    npad = -(-(n + 1) // 2048) * 2048
    nchk = 2
    quant = NW * 128 * nchk
    epad = -(-e_cnt // quant) * quant
    csz = epad // nchk

    z = nxyz[:, 0].astype(jnp.int32)
    xyz = nxyz[:, 1:4].astype(jnp.float32)
    xyz_pad = jnp.zeros((npad, 16), jnp.float32).at[:n, :3].set(xyz)

    a = nbr_list.astype(jnp.int32)
    a0 = jnp.full((epad,), n, jnp.int32).at[:e_cnt].set(a[:, 0])
    a1 = jnp.full((epad,), n, jnp.int32).at[:e_cnt].set(a[:, 1])
    a0c = [a0[i * csz:(i + 1) * csz] for i in range(nchk)]
    a1c = [a1[i * csz:(i + 1) * csz] for i in range(nchk)]
    z_pad = jnp.zeros((npad,), jnp.int32).at[:n].set(z)

    emb = params['emb']
    emb_pad = jnp.zeros((128, emb.shape[1]), jnp.float32).at[:emb.shape[0]].set(emb)

    r = _sc_gather(emb_pad, z_pad, 64)
    dvec = [_sc_dvec(xyz_pad, a0c[i], a1c[i]) for i in range(nchk)]

    offs = jnp.linspace(0.0, _CUTOFF, _NG).astype(jnp.float32)
    width = float(_CUTOFF / (_NG - 1))
    offs = jnp.concatenate([offs, jnp.full((14,), 1e9, jnp.float32)])
    offs = offs.reshape(1, 64)

    efp = params['ef']
    w1 = jnp.zeros((64, efp[0][0].shape[1]), jnp.float32).at[:_NG].set(efp[0][0])
    ea = [_tc_geom(dvec[i], w1, efp[0][1].reshape(1, -1),
                   efp[1][0], efp[1][1].reshape(1, -1), offs, width)
          for i in range(nchk)]
    e = [x[0] for x in ea]
    adj = [x[1] for x in ea]

    for cp in params['convs']:
        rf = _tc_mlp(r, cp['atom_filter'][0], cp['atom_filter'][1])
        ef = [_tc_mlp(e[i], cp['edge_filter'][0], cp['edge_filter'][1])
              for i in range(nchk)]
        sd = [_sc_convmsg(rf, ef[i], a0c[i], a1c[i], npad)
              for i in range(nchk)]
        drp = jnp.concatenate([x[1] for x in sd], axis=0)
        r = _tc_mlp(drp, cp['atom_update'][0], cp['atom_update'][1],
                    res=r, drp_in=True)
        e = [_tc_mlp(sd[i][0], cp['edge_update'][0], cp['edge_update'][1],
                     res=e[i]) for i in range(nchk)]

    facc = []
    for i in range(nchk):
        fpos, fneg = _tc_readout(e[i], adj[i], params['ro'][0], params['ro'][1])
        facc.append(_sc_fscatter(fpos, fneg, a0c[i], a1c[i], npad))
    f_atom = _tc_combine(jnp.concatenate(facc, axis=0))
    return f_atom[:n, :3]


# speedup vs baseline: 1.6451x; 1.0882x over previous
import functools

import jax
import jax.numpy as jnp
from jax import lax
from jax.experimental import pallas as pl
from jax.experimental.pallas import tpu as pltpu
from jax.experimental.pallas import tpu_sc as plsc

NC, NS, LANES = 2, 16, 16
NW = NC * NS

_LN2 = 0.6931471805599453
_CUTOFF = 5.0
_NG = 50
_HW = 64


def _ssp(x):
    return jax.nn.softplus(x) - _LN2


def _sc_mesh():
    return plsc.VectorSubcoreMesh(core_axis_name="c", subcore_axis_name="s")


_SC_CP = pltpu.CompilerParams(use_tc_tiling_on_sc=False)


def _sc_gather(tbl, idx, bsz):
    rows = idx.shape[0]
    d = tbl.shape[1]
    nblk = rows // (NW * bsz)

    @functools.partial(
        pl.kernel,
        out_type=pltpu.HBM((rows, d), jnp.float32),
        mesh=_sc_mesh(),
        scratch_types=[
            pltpu.VMEM((bsz,), jnp.int32),
            pltpu.VMEM((bsz, d), jnp.float32),
            pltpu.SemaphoreType.DMA,
        ],
    )
    def k(tbl_ref, idx_ref, out_ref, idx_v, rows_v, sem):
        wid = lax.axis_index("s") * NC + lax.axis_index("c")

        def body(i, c):
            base = (wid * nblk + i) * bsz
            pltpu.sync_copy(idx_ref.at[pl.ds(base, bsz)], idx_v)
            pltpu.async_copy(tbl_ref.at[idx_v], rows_v, sem).wait()
            pltpu.sync_copy(rows_v, out_ref.at[pl.ds(base, bsz)])
            return c

        lax.fori_loop(0, nblk, body, 0)

    return k(tbl, idx)


def _sc_dvec(xyz_pad, a0, a1):
    rows = a0.shape[0]
    d = xyz_pad.shape[1]
    bsz = 128
    nblk = rows // (NW * bsz)

    @functools.partial(
        pl.kernel,
        out_type=pltpu.HBM((rows, d), jnp.float32),
        mesh=_sc_mesh(),
        compiler_params=_SC_CP,
        scratch_types=[
            pltpu.VMEM((bsz,), jnp.int32),
            pltpu.VMEM((bsz,), jnp.int32),
            pltpu.VMEM((bsz, d), jnp.float32),
            pltpu.VMEM((bsz, d), jnp.float32),
            pltpu.SemaphoreType.DMA,
            pltpu.SemaphoreType.DMA,
        ],
    )
    def k(tbl_ref, a0_ref, a1_ref, out_ref, i0_v, i1_v, r0_v, r1_v, s0, s1):
        wid = lax.axis_index("s") * NC + lax.axis_index("c")

        def body(i, c):
            base = (wid * nblk + i) * bsz
            pltpu.sync_copy(a0_ref.at[pl.ds(base, bsz)], i0_v)
            pltpu.sync_copy(a1_ref.at[pl.ds(base, bsz)], i1_v)
            c0 = pltpu.async_copy(tbl_ref.at[i0_v], r0_v, s0)
            c1 = pltpu.async_copy(tbl_ref.at[i1_v], r1_v, s1)
            c0.wait()
            c1.wait()

            def row(j, cc):
                r0_v[j] = r0_v[j] - r1_v[j]
                return cc

            lax.fori_loop(0, bsz, row, 0)
            pltpu.sync_copy(r0_v, out_ref.at[pl.ds(base, bsz)])
            return c

        lax.fori_loop(0, nblk, body, 0)

    return k(xyz_pad, a0, a1)


def _sc_convmsg(rf, ef, a0, a1, npad):
    rows = a0.shape[0]
    bsz = 64
    d = rf.shape[1]
    nblk = rows // (NW * bsz)
    rps = npad // NS
    nchunk = rps // bsz

    @functools.partial(
        pl.kernel,
        out_type=(
            pltpu.HBM((rows, d), jnp.float32),
            pltpu.HBM((2, npad, d), jnp.float32),
        ),
        mesh=_sc_mesh(),
        compiler_params=_SC_CP,
        scratch_types=[
            pltpu.VMEM((bsz,), jnp.int32),
            pltpu.VMEM((bsz,), jnp.int32),
            pltpu.VMEM((bsz, d), jnp.float32),
            pltpu.VMEM((bsz, d), jnp.float32),
            pltpu.VMEM((bsz, d), jnp.float32),
            pltpu.VMEM((bsz, d), jnp.float32),
            pltpu.VMEM_SHARED((npad, d), jnp.float32),
            pltpu.SemaphoreType.DMA,
            pltpu.SemaphoreType.DMA,
        ],
    )
    def k(rf_ref, ef_ref, a0_ref, a1_ref, s_out, drp_out,
          i0_v, i1_v, ef_v, r0_v, r1_v, s_v, acc, sem0, sem1):
        cid = lax.axis_index("c")
        sid = lax.axis_index("s")
        wid = sid * NC + cid

        def zero_sv(j, c):
            for kk in range(d // 16):
                s_v[j, pl.ds(kk * 16, 16)] = jnp.zeros((16,), jnp.float32)
            return c

        lax.fori_loop(0, bsz, zero_sv, 0)

        def zc(q, c):
            pltpu.sync_copy(s_v, acc.at[pl.ds(sid * rps + q * bsz, bsz)])
            return c

        lax.fori_loop(0, nchunk, zc, 0)
        plsc.subcore_barrier()

        def body(i, c):
            base = (wid * nblk + i) * bsz
            pltpu.sync_copy(a0_ref.at[pl.ds(base, bsz)], i0_v)
            pltpu.sync_copy(a1_ref.at[pl.ds(base, bsz)], i1_v)
            c0 = pltpu.async_copy(rf_ref.at[i0_v], r0_v, sem0)
            c1 = pltpu.async_copy(rf_ref.at[i1_v], r1_v, sem1)
            pltpu.sync_copy(ef_ref.at[pl.ds(base, bsz)], ef_v)
            c0.wait()
            c1.wait()

            def row(j, cc):
                for kk in range(d // 16):
                    dsl = pl.ds(kk * 16, 16)
                    e = ef_v[j, dsl]
                    x0 = r0_v[j, dsl] * e
                    x1 = r1_v[j, dsl] * e
                    r0_v[j, dsl] = x0
                    r1_v[j, dsl] = x1
                    s_v[j, dsl] = x0 + x1
                return cc

            lax.fori_loop(0, bsz, row, 0)
            pltpu.sync_copy(s_v, s_out.at[pl.ds(base, bsz)])
            pltpu.sync_copy(r0_v, acc.at[i1_v], add=True)
            pltpu.sync_copy(r1_v, acc.at[i0_v], add=True)
            return c

        lax.fori_loop(0, nblk, body, 0)
        plsc.subcore_barrier()

        def dump(q, c):
            ro = sid * rps + q * bsz
            pltpu.sync_copy(acc.at[pl.ds(ro, bsz)], s_v)
            pltpu.sync_copy(s_v, drp_out.at[cid, pl.ds(ro, bsz)])
            return c

        lax.fori_loop(0, nchunk, dump, 0)

    return k(rf, ef, a0, a1)


def _sc_fscatter(fpos, fneg, a0, a1, npad):
    rows = a0.shape[0]
    d = fpos.shape[1]
    bsz = 128
    nblk = rows // (NW * bsz)
    rps = npad // NS
    nchunk = rps // bsz

    @functools.partial(
        pl.kernel,
        out_type=pltpu.HBM((2, npad, d), jnp.float32),
        mesh=_sc_mesh(),
        compiler_params=_SC_CP,
        scratch_types=[
            pltpu.VMEM((bsz,), jnp.int32),
            pltpu.VMEM((bsz,), jnp.int32),
            pltpu.VMEM((bsz, d), jnp.float32),
            pltpu.VMEM((bsz, d), jnp.float32),
            pltpu.VMEM_SHARED((npad, d), jnp.float32),
        ],
    )
    def k(fp_ref, fn_ref, a0_ref, a1_ref, out_ref,
          i0_v, i1_v, f0_v, f1_v, acc):
        cid = lax.axis_index("c")
        sid = lax.axis_index("s")
        wid = sid * NC + cid

        def zrow(j, c):
            f0_v[j] = jnp.zeros((d,), jnp.float32)
            return c

        lax.fori_loop(0, bsz, zrow, 0)

        def zc(q, c):
            pltpu.sync_copy(f0_v, acc.at[pl.ds(sid * rps + q * bsz, bsz)])
            return c

        lax.fori_loop(0, nchunk, zc, 0)
        plsc.subcore_barrier()

        def body(i, c):
            base = (wid * nblk + i) * bsz
            pltpu.sync_copy(a0_ref.at[pl.ds(base, bsz)], i0_v)
            pltpu.sync_copy(a1_ref.at[pl.ds(base, bsz)], i1_v)
            pltpu.sync_copy(fp_ref.at[pl.ds(base, bsz)], f0_v)
            pltpu.sync_copy(fn_ref.at[pl.ds(base, bsz)], f1_v)
            pltpu.sync_copy(f0_v, acc.at[i0_v], add=True)
            pltpu.sync_copy(f1_v, acc.at[i1_v], add=True)
            return c

        lax.fori_loop(0, nblk, body, 0)
        plsc.subcore_barrier()

        def dump(q, c):
            ro = sid * rps + q * bsz
            pltpu.sync_copy(acc.at[pl.ds(ro, bsz)], f0_v)
            pltpu.sync_copy(f0_v, out_ref.at[cid, pl.ds(ro, bsz)])
            return c

        lax.fori_loop(0, nchunk, dump, 0)

    return k(fpos, fneg, a0, a1)


_BLK = 512


def _row_spec(blk, d):
    return pl.BlockSpec((blk, d), lambda i: (i, 0))


def _full_spec(shape):
    nd = len(shape)
    return pl.BlockSpec(shape, lambda i: (0,) * nd)


def _tc_params():
    return pltpu.CompilerParams(dimension_semantics=("parallel",))


def _tc_geom(dvec, w1, b1, w2, b2, offs, width):
    rows, dcoord = dvec.shape
    grid = rows // _BLK

    def body(dv_ref, offs_ref, w1_ref, b1_ref, w2_ref, b2_ref, e_ref, adj_ref):
        dv = dv_ref[...]
        d2 = jnp.sum(dv * dv, axis=1, keepdims=True)
        dis = jnp.sqrt(d2)
        adj_ref[...] = dv / dis
        x = (dis - offs_ref[...]) / width
        e0 = jnp.exp(-0.5 * x * x)
        h = _ssp(jnp.dot(e0, w1_ref[...], preferred_element_type=jnp.float32)
                 + b1_ref[...])
        e_ref[...] = (jnp.dot(h, w2_ref[...], preferred_element_type=jnp.float32)
                      + b2_ref[...])

    return pl.pallas_call(
        body,
        grid=(grid,),
        in_specs=[
            _row_spec(_BLK, dcoord),
            _full_spec(offs.shape),
            _full_spec(w1.shape),
            _full_spec(b1.shape),
            _full_spec(w2.shape),
            _full_spec(b2.shape),
        ],
        out_specs=[_row_spec(_BLK, 128), _row_spec(_BLK, dcoord)],
        out_shape=[
            jax.ShapeDtypeStruct((rows, 128), jnp.float32),
            jax.ShapeDtypeStruct((rows, dcoord), jnp.float32),
        ],
        compiler_params=_tc_params(),
    )(dvec, offs, w1, b1, w2, b2)


def _tc_mlp(x, p1, p2, res=None, drp_in=False):
    rows = x.shape[1] if drp_in else x.shape[0]
    grid = rows // _BLK
    w1, b1 = p1[0], p1[1].reshape(1, -1)
    w2, b2 = p2[0], p2[1].reshape(1, -1)
    dout = w2.shape[1]

    nsum = x.shape[0] if drp_in else 0

    def body(*refs):
        refs = list(refs)
        x_ref = refs.pop(0)
        w1_ref, b1_ref, w2_ref, b2_ref = refs[:4]
        refs = refs[4:]
        r_ref = refs.pop(0) if res is not None else None
        o_ref = refs.pop(0)
        if drp_in:
            xv = x_ref[0]
            for q in range(1, nsum):
                xv = xv + x_ref[q]
        else:
            xv = x_ref[...]
        h = _ssp(jnp.dot(xv, w1_ref[...],
                         preferred_element_type=jnp.float32) + b1_ref[...])
        o = (jnp.dot(h, w2_ref[...], preferred_element_type=jnp.float32)
             + b2_ref[...])
        if res is not None:
            o = o + r_ref[...]
        o_ref[...] = o

    if drp_in:
        x_spec = pl.BlockSpec((nsum, _BLK, x.shape[2]),
                              lambda i: (0, i, 0))
    else:
        x_spec = _row_spec(_BLK, x.shape[1])

    in_specs = [
        x_spec,
        _full_spec(w1.shape),
        _full_spec(b1.shape),
        _full_spec(w2.shape),
        _full_spec(b2.shape),
    ]
    args = [x, w1, b1, w2, b2]
    if res is not None:
        in_specs.append(_row_spec(_BLK, dout))
        args.append(res)

    return pl.pallas_call(
        body,
        grid=(grid,),
        in_specs=in_specs,
        out_specs=_row_spec(_BLK, dout),
        out_shape=jax.ShapeDtypeStruct((rows, dout), jnp.float32),
        compiler_params=_tc_params(),
    )(*args)


def _tc_readout(e, adj, p1, p2):
    rows, dcoord = adj.shape
    grid = rows // _BLK
    w1, b1 = p1[0], p1[1].reshape(1, -1)
    w2, b2 = p2[0], p2[1].reshape(1, -1)

    def body(e_ref, a_ref, w1_ref, b1_ref, w2_ref, b2_ref, fp_ref, fn_ref):
        h = _ssp(jnp.dot(e_ref[...], w1_ref[...],
                         preferred_element_type=jnp.float32) + b1_ref[...])
        val = (jnp.dot(h, w2_ref[...], preferred_element_type=jnp.float32)
               + b2_ref[...])
        fp = val * a_ref[...]
        fp_ref[...] = fp
        fn_ref[...] = -fp

    return pl.pallas_call(
        body,
        grid=(grid,),
        in_specs=[
            _row_spec(_BLK, 128),
            _row_spec(_BLK, dcoord),
            _full_spec(w1.shape),
            _full_spec(b1.shape),
            _full_spec(w2.shape),
            _full_spec(b2.shape),
        ],
        out_specs=[_row_spec(_BLK, dcoord), _row_spec(_BLK, dcoord)],
        out_shape=[
            jax.ShapeDtypeStruct((rows, dcoord), jnp.float32),
            jax.ShapeDtypeStruct((rows, dcoord), jnp.float32),
        ],
        compiler_params=_tc_params(),
    )(e, adj, w1, b1, w2, b2)


def _tc_combine(facc):
    k, rows, d = facc.shape
    grid = rows // _BLK

    def body(f_ref, o_ref):
        o = f_ref[0]
        for q in range(1, k):
            o = o + f_ref[q]
        o_ref[...] = o

    return pl.pallas_call(
        body,
        grid=(grid,),
        in_specs=[pl.BlockSpec((k, _BLK, d), lambda i: (0, i, 0))],
        out_specs=_row_spec(_BLK, d),
        out_shape=jax.ShapeDtypeStruct((rows, d), jnp.float32),
        compiler_params=_tc_params(),
    )(facc)


def kernel(nxyz, nbr_list, params):
    n = nxyz.shape[0]
    e_cnt = nbr_list.shape[0]
    npad = -(-(n + 1) // 2048) * 2048
    nchk = 4
    quant = NW * 128 * nchk
    epad = -(-e_cnt // quant) * quant
    csz = epad // nchk

    z = nxyz[:, 0].astype(jnp.int32)
    xyz = nxyz[:, 1:4].astype(jnp.float32)
    xyz_pad = jnp.zeros((npad, 16), jnp.float32).at[:n, :3].set(xyz)

    a = nbr_list.astype(jnp.int32)
    a0 = jnp.full((epad,), n, jnp.int32).at[:e_cnt].set(a[:, 0])
    a1 = jnp.full((epad,), n, jnp.int32).at[:e_cnt].set(a[:, 1])
    a0c = [a0[i * csz:(i + 1) * csz] for i in range(nchk)]
    a1c = [a1[i * csz:(i + 1) * csz] for i in range(nchk)]
    z_pad = jnp.zeros((npad,), jnp.int32).at[:n].set(z)

    emb = params['emb']
    emb_pad = jnp.zeros((128, emb.shape[1]), jnp.float32).at[:emb.shape[0]].set(emb)

    r = _sc_gather(emb_pad, z_pad, 64)
    dvec = [_sc_dvec(xyz_pad, a0c[i], a1c[i]) for i in range(nchk)]

    offs = jnp.linspace(0.0, _CUTOFF, _NG).astype(jnp.float32)
    width = float(_CUTOFF / (_NG - 1))
    offs = jnp.concatenate([offs, jnp.full((14,), 1e9, jnp.float32)])
    offs = offs.reshape(1, 64)

    efp = params['ef']
    w1 = jnp.zeros((64, efp[0][0].shape[1]), jnp.float32).at[:_NG].set(efp[0][0])
    ea = [_tc_geom(dvec[i], w1, efp[0][1].reshape(1, -1),
                   efp[1][0], efp[1][1].reshape(1, -1), offs, width)
          for i in range(nchk)]
    e = [x[0] for x in ea]
    adj = [x[1] for x in ea]

    for cp in params['convs']:
        rf = _tc_mlp(r, cp['atom_filter'][0], cp['atom_filter'][1])
        ef = [_tc_mlp(e[i], cp['edge_filter'][0], cp['edge_filter'][1])
              for i in range(nchk)]
        sd = [_sc_convmsg(rf, ef[i], a0c[i], a1c[i], npad)
              for i in range(nchk)]
        drp = jnp.concatenate([x[1] for x in sd], axis=0)
        r = _tc_mlp(drp, cp['atom_update'][0], cp['atom_update'][1],
                    res=r, drp_in=True)
        e = [_tc_mlp(sd[i][0], cp['edge_update'][0], cp['edge_update'][1],
                     res=e[i]) for i in range(nchk)]

    facc = []
    for i in range(nchk):
        fpos, fneg = _tc_readout(e[i], adj[i], params['ro'][0], params['ro'][1])
        facc.append(_sc_fscatter(fpos, fneg, a0c[i], a1c[i], npad))
    f_atom = _tc_combine(jnp.concatenate(facc, axis=0))
    return f_atom[:n, :3]


# speedup vs baseline: 1.6496x; 1.0027x over previous
import functools

import jax
import jax.numpy as jnp
from jax import lax
from jax.experimental import pallas as pl
from jax.experimental.pallas import tpu as pltpu
from jax.experimental.pallas import tpu_sc as plsc

NC, NS, LANES = 2, 16, 16
NW = NC * NS

_LN2 = 0.6931471805599453
_CUTOFF = 5.0
_NG = 50
_HW = 64


def _ssp(x):
    return jax.nn.softplus(x) - _LN2


def _sc_mesh():
    return plsc.VectorSubcoreMesh(core_axis_name="c", subcore_axis_name="s")


_SC_CP = pltpu.CompilerParams(use_tc_tiling_on_sc=False)


def _sc_gather(tbl, idx, bsz):
    rows = idx.shape[0]
    d = tbl.shape[1]
    nblk = rows // (NW * bsz)

    @functools.partial(
        pl.kernel,
        out_type=pltpu.HBM((rows, d), jnp.float32),
        mesh=_sc_mesh(),
        scratch_types=[
            pltpu.VMEM((bsz,), jnp.int32),
            pltpu.VMEM((bsz, d), jnp.float32),
            pltpu.SemaphoreType.DMA,
        ],
    )
    def k(tbl_ref, idx_ref, out_ref, idx_v, rows_v, sem):
        wid = lax.axis_index("s") * NC + lax.axis_index("c")

        def body(i, c):
            base = (wid * nblk + i) * bsz
            pltpu.sync_copy(idx_ref.at[pl.ds(base, bsz)], idx_v)
            pltpu.async_copy(tbl_ref.at[idx_v], rows_v, sem).wait()
            pltpu.sync_copy(rows_v, out_ref.at[pl.ds(base, bsz)])
            return c

        lax.fori_loop(0, nblk, body, 0)

    return k(tbl, idx)


def _sc_dvec(xyz_pad, a0, a1):
    rows = a0.shape[0]
    d = xyz_pad.shape[1]
    bsz = 128
    nblk = rows // (NW * bsz)

    @functools.partial(
        pl.kernel,
        out_type=pltpu.HBM((rows, d), jnp.float32),
        mesh=_sc_mesh(),
        compiler_params=_SC_CP,
        scratch_types=[
            pltpu.VMEM((bsz,), jnp.int32),
            pltpu.VMEM((bsz,), jnp.int32),
            pltpu.VMEM((bsz, d), jnp.float32),
            pltpu.VMEM((bsz, d), jnp.float32),
            pltpu.SemaphoreType.DMA,
            pltpu.SemaphoreType.DMA,
        ],
    )
    def k(tbl_ref, a0_ref, a1_ref, out_ref, i0_v, i1_v, r0_v, r1_v, s0, s1):
        wid = lax.axis_index("s") * NC + lax.axis_index("c")

        def body(i, c):
            base = (wid * nblk + i) * bsz
            pltpu.sync_copy(a0_ref.at[pl.ds(base, bsz)], i0_v)
            pltpu.sync_copy(a1_ref.at[pl.ds(base, bsz)], i1_v)
            c0 = pltpu.async_copy(tbl_ref.at[i0_v], r0_v, s0)
            c1 = pltpu.async_copy(tbl_ref.at[i1_v], r1_v, s1)
            c0.wait()
            c1.wait()

            def row(j, cc):
                r0_v[j] = r0_v[j] - r1_v[j]
                return cc

            lax.fori_loop(0, bsz, row, 0)
            pltpu.sync_copy(r0_v, out_ref.at[pl.ds(base, bsz)])
            return c

        lax.fori_loop(0, nblk, body, 0)

    return k(xyz_pad, a0, a1)


def _sc_convmsg(rf, ef, a0, a1, npad):
    rows = a0.shape[0]
    bsz = 64
    d = rf.shape[1]
    nblk = rows // (NW * bsz)
    rps = npad // NS
    nchunk = rps // bsz

    @functools.partial(
        pl.kernel,
        out_type=(
            pltpu.HBM((rows, d), jnp.float32),
            pltpu.HBM((2, npad, d), jnp.float32),
        ),
        mesh=_sc_mesh(),
        compiler_params=_SC_CP,
        scratch_types=[
            pltpu.VMEM((bsz,), jnp.int32),
            pltpu.VMEM((bsz,), jnp.int32),
            pltpu.VMEM((bsz, d), jnp.float32),
            pltpu.VMEM((bsz, d), jnp.float32),
            pltpu.VMEM((bsz, d), jnp.float32),
            pltpu.VMEM((bsz, d), jnp.float32),
            pltpu.VMEM_SHARED((npad, d), jnp.float32),
            pltpu.SemaphoreType.DMA,
            pltpu.SemaphoreType.DMA,
        ],
    )
    def k(rf_ref, ef_ref, a0_ref, a1_ref, s_out, drp_out,
          i0_v, i1_v, ef_v, r0_v, r1_v, s_v, acc, sem0, sem1):
        cid = lax.axis_index("c")
        sid = lax.axis_index("s")
        wid = sid * NC + cid

        def zero_sv(j, c):
            for kk in range(d // 16):
                s_v[j, pl.ds(kk * 16, 16)] = jnp.zeros((16,), jnp.float32)
            return c

        lax.fori_loop(0, bsz, zero_sv, 0)

        def zc(q, c):
            pltpu.sync_copy(s_v, acc.at[pl.ds(sid * rps + q * bsz, bsz)])
            return c

        lax.fori_loop(0, nchunk, zc, 0)
        plsc.subcore_barrier()

        def body(i, c):
            base = (wid * nblk + i) * bsz
            pltpu.sync_copy(a0_ref.at[pl.ds(base, bsz)], i0_v)
            pltpu.sync_copy(a1_ref.at[pl.ds(base, bsz)], i1_v)
            c0 = pltpu.async_copy(rf_ref.at[i0_v], r0_v, sem0)
            c1 = pltpu.async_copy(rf_ref.at[i1_v], r1_v, sem1)
            pltpu.sync_copy(ef_ref.at[pl.ds(base, bsz)], ef_v)
            c0.wait()
            c1.wait()

            def row(j, cc):
                for kk in range(d // 16):
                    dsl = pl.ds(kk * 16, 16)
                    e = ef_v[j, dsl]
                    x0 = r0_v[j, dsl] * e
                    x1 = r1_v[j, dsl] * e
                    r0_v[j, dsl] = x0
                    r1_v[j, dsl] = x1
                    s_v[j, dsl] = x0 + x1
                return cc

            lax.fori_loop(0, bsz, row, 0)
            pltpu.sync_copy(s_v, s_out.at[pl.ds(base, bsz)])
            pltpu.sync_copy(r0_v, acc.at[i1_v], add=True)
            pltpu.sync_copy(r1_v, acc.at[i0_v], add=True)
            return c

        lax.fori_loop(0, nblk, body, 0)
        plsc.subcore_barrier()

        def dump(q, c):
            ro = sid * rps + q * bsz
            pltpu.sync_copy(acc.at[pl.ds(ro, bsz)], s_v)
            pltpu.sync_copy(s_v, drp_out.at[cid, pl.ds(ro, bsz)])
            return c

        lax.fori_loop(0, nchunk, dump, 0)

    return k(rf, ef, a0, a1)


def _sc_fscatter(fpos, fneg, a0, a1, npad):
    rows = a0.shape[0]
    d = fpos.shape[1]
    bsz = 128
    nblk = rows // (NW * bsz)
    rps = npad // NS
    nchunk = rps // bsz

    @functools.partial(
        pl.kernel,
        out_type=pltpu.HBM((2, npad, d), jnp.float32),
        mesh=_sc_mesh(),
        compiler_params=_SC_CP,
        scratch_types=[
            pltpu.VMEM((bsz,), jnp.int32),
            pltpu.VMEM((bsz,), jnp.int32),
            pltpu.VMEM((bsz, d), jnp.float32),
            pltpu.VMEM((bsz, d), jnp.float32),
            pltpu.VMEM_SHARED((npad, d), jnp.float32),
        ],
    )
    def k(fp_ref, fn_ref, a0_ref, a1_ref, out_ref,
          i0_v, i1_v, f0_v, f1_v, acc):
        cid = lax.axis_index("c")
        sid = lax.axis_index("s")
        wid = sid * NC + cid

        def zrow(j, c):
            f0_v[j] = jnp.zeros((d,), jnp.float32)
            return c

        lax.fori_loop(0, bsz, zrow, 0)

        def zc(q, c):
            pltpu.sync_copy(f0_v, acc.at[pl.ds(sid * rps + q * bsz, bsz)])
            return c

        lax.fori_loop(0, nchunk, zc, 0)
        plsc.subcore_barrier()

        def body(i, c):
            base = (wid * nblk + i) * bsz
            pltpu.sync_copy(a0_ref.at[pl.ds(base, bsz)], i0_v)
            pltpu.sync_copy(a1_ref.at[pl.ds(base, bsz)], i1_v)
            pltpu.sync_copy(fp_ref.at[pl.ds(base, bsz)], f0_v)
            pltpu.sync_copy(fn_ref.at[pl.ds(base, bsz)], f1_v)
            pltpu.sync_copy(f0_v, acc.at[i0_v], add=True)
            pltpu.sync_copy(f1_v, acc.at[i1_v], add=True)
            return c

        lax.fori_loop(0, nblk, body, 0)
        plsc.subcore_barrier()

        def dump(q, c):
            ro = sid * rps + q * bsz
            pltpu.sync_copy(acc.at[pl.ds(ro, bsz)], f0_v)
            pltpu.sync_copy(f0_v, out_ref.at[cid, pl.ds(ro, bsz)])
            return c

        lax.fori_loop(0, nchunk, dump, 0)

    return k(fpos, fneg, a0, a1)


_BLK = 512


def _row_spec(blk, d):
    return pl.BlockSpec((blk, d), lambda i: (i, 0))


def _full_spec(shape):
    nd = len(shape)
    return pl.BlockSpec(shape, lambda i: (0,) * nd)


def _tc_params():
    return pltpu.CompilerParams(dimension_semantics=("parallel",))


def _tc_geom(dvec, w1, b1, w2, b2, offs, width):
    rows, dcoord = dvec.shape
    grid = rows // _BLK

    def body(dv_ref, offs_ref, w1_ref, b1_ref, w2_ref, b2_ref, e_ref, adj_ref):
        dv = dv_ref[...]
        d2 = jnp.sum(dv * dv, axis=1, keepdims=True)
        dis = jnp.sqrt(d2)
        adj_ref[...] = dv / dis
        x = (dis - offs_ref[...]) / width
        e0 = jnp.exp(-0.5 * x * x)
        h = _ssp(jnp.dot(e0, w1_ref[...], preferred_element_type=jnp.float32)
                 + b1_ref[...])
        e_ref[...] = (jnp.dot(h, w2_ref[...], preferred_element_type=jnp.float32)
                      + b2_ref[...])

    return pl.pallas_call(
        body,
        grid=(grid,),
        in_specs=[
            _row_spec(_BLK, dcoord),
            _full_spec(offs.shape),
            _full_spec(w1.shape),
            _full_spec(b1.shape),
            _full_spec(w2.shape),
            _full_spec(b2.shape),
        ],
        out_specs=[_row_spec(_BLK, 128), _row_spec(_BLK, dcoord)],
        out_shape=[
            jax.ShapeDtypeStruct((rows, 128), jnp.float32),
            jax.ShapeDtypeStruct((rows, dcoord), jnp.float32),
        ],
        compiler_params=_tc_params(),
    )(dvec, offs, w1, b1, w2, b2)


def _tc_mlp(x, p1, p2, res=None, drp_in=False):
    rows = x.shape[1] if drp_in else x.shape[0]
    grid = rows // _BLK
    w1, b1 = p1[0], p1[1].reshape(1, -1)
    w2, b2 = p2[0], p2[1].reshape(1, -1)
    dout = w2.shape[1]

    nsum = x.shape[0] if drp_in else 0

    def body(*refs):
        refs = list(refs)
        x_ref = refs.pop(0)
        w1_ref, b1_ref, w2_ref, b2_ref = refs[:4]
        refs = refs[4:]
        r_ref = refs.pop(0) if res is not None else None
        o_ref = refs.pop(0)
        if drp_in:
            xv = x_ref[0]
            for q in range(1, nsum):
                xv = xv + x_ref[q]
        else:
            xv = x_ref[...]
        h = _ssp(jnp.dot(xv, w1_ref[...],
                         preferred_element_type=jnp.float32) + b1_ref[...])
        o = (jnp.dot(h, w2_ref[...], preferred_element_type=jnp.float32)
             + b2_ref[...])
        if res is not None:
            o = o + r_ref[...]
        o_ref[...] = o

    if drp_in:
        x_spec = pl.BlockSpec((nsum, _BLK, x.shape[2]),
                              lambda i: (0, i, 0))
    else:
        x_spec = _row_spec(_BLK, x.shape[1])

    in_specs = [
        x_spec,
        _full_spec(w1.shape),
        _full_spec(b1.shape),
        _full_spec(w2.shape),
        _full_spec(b2.shape),
    ]
    args = [x, w1, b1, w2, b2]
    if res is not None:
        in_specs.append(_row_spec(_BLK, dout))
        args.append(res)

    return pl.pallas_call(
        body,
        grid=(grid,),
        in_specs=in_specs,
        out_specs=_row_spec(_BLK, dout),
        out_shape=jax.ShapeDtypeStruct((rows, dout), jnp.float32),
        compiler_params=_tc_params(),
    )(*args)


def _tc_readout(e, adj, p1, p2):
    rows, dcoord = adj.shape
    grid = rows // _BLK
    w1, b1 = p1[0], p1[1].reshape(1, -1)
    w2, b2 = p2[0], p2[1].reshape(1, -1)

    def body(e_ref, a_ref, w1_ref, b1_ref, w2_ref, b2_ref, fp_ref, fn_ref):
        h = _ssp(jnp.dot(e_ref[...], w1_ref[...],
                         preferred_element_type=jnp.float32) + b1_ref[...])
        val = (jnp.dot(h, w2_ref[...], preferred_element_type=jnp.float32)
               + b2_ref[...])
        fp = val * a_ref[...]
        fp_ref[...] = fp
        fn_ref[...] = -fp

    return pl.pallas_call(
        body,
        grid=(grid,),
        in_specs=[
            _row_spec(_BLK, 128),
            _row_spec(_BLK, dcoord),
            _full_spec(w1.shape),
            _full_spec(b1.shape),
            _full_spec(w2.shape),
            _full_spec(b2.shape),
        ],
        out_specs=[_row_spec(_BLK, dcoord), _row_spec(_BLK, dcoord)],
        out_shape=[
            jax.ShapeDtypeStruct((rows, dcoord), jnp.float32),
            jax.ShapeDtypeStruct((rows, dcoord), jnp.float32),
        ],
        compiler_params=_tc_params(),
    )(e, adj, w1, b1, w2, b2)


def _tc_combine(facc):
    k, rows, d = facc.shape
    grid = rows // _BLK

    def body(f_ref, o_ref):
        o = f_ref[0]
        for q in range(1, k):
            o = o + f_ref[q]
        o_ref[...] = o

    return pl.pallas_call(
        body,
        grid=(grid,),
        in_specs=[pl.BlockSpec((k, _BLK, d), lambda i: (0, i, 0))],
        out_specs=_row_spec(_BLK, d),
        out_shape=jax.ShapeDtypeStruct((rows, d), jnp.float32),
        compiler_params=_tc_params(),
    )(facc)


def kernel(nxyz, nbr_list, params):
    n = nxyz.shape[0]
    e_cnt = nbr_list.shape[0]
    npad = -(-(n + 1) // 2048) * 2048
    nchk = 8
    quant = NW * 128 * nchk
    epad = -(-e_cnt // quant) * quant
    csz = epad // nchk

    z = nxyz[:, 0].astype(jnp.int32)
    xyz = nxyz[:, 1:4].astype(jnp.float32)
    xyz_pad = jnp.zeros((npad, 16), jnp.float32).at[:n, :3].set(xyz)

    a = nbr_list.astype(jnp.int32)
    a0 = jnp.full((epad,), n, jnp.int32).at[:e_cnt].set(a[:, 0])
    a1 = jnp.full((epad,), n, jnp.int32).at[:e_cnt].set(a[:, 1])
    a0c = [a0[i * csz:(i + 1) * csz] for i in range(nchk)]
    a1c = [a1[i * csz:(i + 1) * csz] for i in range(nchk)]
    z_pad = jnp.zeros((npad,), jnp.int32).at[:n].set(z)

    emb = params['emb']
    emb_pad = jnp.zeros((128, emb.shape[1]), jnp.float32).at[:emb.shape[0]].set(emb)

    r = _sc_gather(emb_pad, z_pad, 64)
    dvec = [_sc_dvec(xyz_pad, a0c[i], a1c[i]) for i in range(nchk)]

    offs = jnp.linspace(0.0, _CUTOFF, _NG).astype(jnp.float32)
    width = float(_CUTOFF / (_NG - 1))
    offs = jnp.concatenate([offs, jnp.full((14,), 1e9, jnp.float32)])
    offs = offs.reshape(1, 64)

    efp = params['ef']
    w1 = jnp.zeros((64, efp[0][0].shape[1]), jnp.float32).at[:_NG].set(efp[0][0])
    ea = [_tc_geom(dvec[i], w1, efp[0][1].reshape(1, -1),
                   efp[1][0], efp[1][1].reshape(1, -1), offs, width)
          for i in range(nchk)]
    e = [x[0] for x in ea]
    adj = [x[1] for x in ea]

    for cp in params['convs']:
        rf = _tc_mlp(r, cp['atom_filter'][0], cp['atom_filter'][1])
        ef = [_tc_mlp(e[i], cp['edge_filter'][0], cp['edge_filter'][1])
              for i in range(nchk)]
        sd = [_sc_convmsg(rf, ef[i], a0c[i], a1c[i], npad)
              for i in range(nchk)]
        drp = jnp.concatenate([x[1] for x in sd], axis=0)
        r = _tc_mlp(drp, cp['atom_update'][0], cp['atom_update'][1],
                    res=r, drp_in=True)
        e = [_tc_mlp(sd[i][0], cp['edge_update'][0], cp['edge_update'][1],
                     res=e[i]) for i in range(nchk)]

    facc = []
    for i in range(nchk):
        fpos, fneg = _tc_readout(e[i], adj[i], params['ro'][0], params['ro'][1])
        facc.append(_sc_fscatter(fpos, fneg, a0c[i], a1c[i], npad))
    f_atom = _tc_combine(jnp.concatenate(facc, axis=0))
    return f_atom[:n, :3]


# speedup vs baseline: 1.6543x; 1.0029x over previous
import functools

import jax
import jax.numpy as jnp
from jax import lax
from jax.experimental import pallas as pl
from jax.experimental.pallas import tpu as pltpu
from jax.experimental.pallas import tpu_sc as plsc

NC, NS, LANES = 2, 16, 16
NW = NC * NS

_LN2 = 0.6931471805599453
_CUTOFF = 5.0
_NG = 50
_HW = 64


def _ssp(x):
    return jax.nn.softplus(x) - _LN2


def _sc_mesh():
    return plsc.VectorSubcoreMesh(core_axis_name="c", subcore_axis_name="s")


_SC_CP = pltpu.CompilerParams(use_tc_tiling_on_sc=False)


def _sc_gather(tbl, idx, bsz):
    rows = idx.shape[0]
    d = tbl.shape[1]
    nblk = rows // (NW * bsz)

    @functools.partial(
        pl.kernel,
        out_type=pltpu.HBM((rows, d), jnp.float32),
        mesh=_sc_mesh(),
        scratch_types=[
            pltpu.VMEM((bsz,), jnp.int32),
            pltpu.VMEM((bsz, d), jnp.float32),
            pltpu.SemaphoreType.DMA,
        ],
    )
    def k(tbl_ref, idx_ref, out_ref, idx_v, rows_v, sem):
        wid = lax.axis_index("s") * NC + lax.axis_index("c")

        def body(i, c):
            base = (wid * nblk + i) * bsz
            pltpu.sync_copy(idx_ref.at[pl.ds(base, bsz)], idx_v)
            pltpu.async_copy(tbl_ref.at[idx_v], rows_v, sem).wait()
            pltpu.sync_copy(rows_v, out_ref.at[pl.ds(base, bsz)])
            return c

        lax.fori_loop(0, nblk, body, 0)

    return k(tbl, idx)


def _sc_dvec(xyz_pad, a0, a1):
    rows = a0.shape[0]
    d = xyz_pad.shape[1]
    bsz = 128
    nblk = rows // (NW * bsz)

    @functools.partial(
        pl.kernel,
        out_type=pltpu.HBM((rows, d), jnp.float32),
        mesh=_sc_mesh(),
        compiler_params=_SC_CP,
        scratch_types=[
            pltpu.VMEM((bsz,), jnp.int32),
            pltpu.VMEM((bsz,), jnp.int32),
            pltpu.VMEM((bsz, d), jnp.float32),
            pltpu.VMEM((bsz, d), jnp.float32),
            pltpu.SemaphoreType.DMA,
            pltpu.SemaphoreType.DMA,
        ],
    )
    def k(tbl_ref, a0_ref, a1_ref, out_ref, i0_v, i1_v, r0_v, r1_v, s0, s1):
        wid = lax.axis_index("s") * NC + lax.axis_index("c")

        def body(i, c):
            base = (wid * nblk + i) * bsz
            pltpu.sync_copy(a0_ref.at[pl.ds(base, bsz)], i0_v)
            pltpu.sync_copy(a1_ref.at[pl.ds(base, bsz)], i1_v)
            c0 = pltpu.async_copy(tbl_ref.at[i0_v], r0_v, s0)
            c1 = pltpu.async_copy(tbl_ref.at[i1_v], r1_v, s1)
            c0.wait()
            c1.wait()

            @functools.partial(plsc.parallel_loop, 0, bsz, unroll=4)
            def row(j):
                r0_v[j] = r0_v[j] - r1_v[j]
            pltpu.sync_copy(r0_v, out_ref.at[pl.ds(base, bsz)])
            return c

        lax.fori_loop(0, nblk, body, 0)

    return k(xyz_pad, a0, a1)


def _sc_convmsg(rf, ef, a0, a1, npad):
    rows = a0.shape[0]
    bsz = 64
    d = rf.shape[1]
    nblk = rows // (NW * bsz)
    rps = npad // NS
    nchunk = rps // bsz

    @functools.partial(
        pl.kernel,
        out_type=(
            pltpu.HBM((rows, d), jnp.float32),
            pltpu.HBM((2, npad, d), jnp.float32),
        ),
        mesh=_sc_mesh(),
        compiler_params=_SC_CP,
        scratch_types=[
            pltpu.VMEM((bsz,), jnp.int32),
            pltpu.VMEM((bsz,), jnp.int32),
            pltpu.VMEM((bsz, d), jnp.float32),
            pltpu.VMEM((bsz, d), jnp.float32),
            pltpu.VMEM((bsz, d), jnp.float32),
            pltpu.VMEM((bsz, d), jnp.float32),
            pltpu.VMEM_SHARED((npad, d), jnp.float32),
            pltpu.SemaphoreType.DMA,
            pltpu.SemaphoreType.DMA,
        ],
    )
    def k(rf_ref, ef_ref, a0_ref, a1_ref, s_out, drp_out,
          i0_v, i1_v, ef_v, r0_v, r1_v, s_v, acc, sem0, sem1):
        cid = lax.axis_index("c")
        sid = lax.axis_index("s")
        wid = sid * NC + cid

        def zero_sv(j, c):
            for kk in range(d // 16):
                s_v[j, pl.ds(kk * 16, 16)] = jnp.zeros((16,), jnp.float32)
            return c

        lax.fori_loop(0, bsz, zero_sv, 0)

        def zc(q, c):
            pltpu.sync_copy(s_v, acc.at[pl.ds(sid * rps + q * bsz, bsz)])
            return c

        lax.fori_loop(0, nchunk, zc, 0)
        plsc.subcore_barrier()

        def body(i, c):
            base = (wid * nblk + i) * bsz
            pltpu.sync_copy(a0_ref.at[pl.ds(base, bsz)], i0_v)
            pltpu.sync_copy(a1_ref.at[pl.ds(base, bsz)], i1_v)
            c0 = pltpu.async_copy(rf_ref.at[i0_v], r0_v, sem0)
            c1 = pltpu.async_copy(rf_ref.at[i1_v], r1_v, sem1)
            pltpu.sync_copy(ef_ref.at[pl.ds(base, bsz)], ef_v)
            c0.wait()
            c1.wait()

            @functools.partial(plsc.parallel_loop, 0, bsz, unroll=4)
            def row(j):
                for kk in range(d // 16):
                    dsl = pl.ds(kk * 16, 16)
                    e = ef_v[j, dsl]
                    x0 = r0_v[j, dsl] * e
                    x1 = r1_v[j, dsl] * e
                    r0_v[j, dsl] = x0
                    r1_v[j, dsl] = x1
                    s_v[j, dsl] = x0 + x1
            pltpu.sync_copy(s_v, s_out.at[pl.ds(base, bsz)])
            pltpu.sync_copy(r0_v, acc.at[i1_v], add=True)
            pltpu.sync_copy(r1_v, acc.at[i0_v], add=True)
            return c

        lax.fori_loop(0, nblk, body, 0)
        plsc.subcore_barrier()

        def dump(q, c):
            ro = sid * rps + q * bsz
            pltpu.sync_copy(acc.at[pl.ds(ro, bsz)], s_v)
            pltpu.sync_copy(s_v, drp_out.at[cid, pl.ds(ro, bsz)])
            return c

        lax.fori_loop(0, nchunk, dump, 0)

    return k(rf, ef, a0, a1)


def _sc_fscatter(fpos, fneg, a0, a1, npad):
    rows = a0.shape[0]
    d = fpos.shape[1]
    bsz = 128
    nblk = rows // (NW * bsz)
    rps = npad // NS
    nchunk = rps // bsz

    @functools.partial(
        pl.kernel,
        out_type=pltpu.HBM((2, npad, d), jnp.float32),
        mesh=_sc_mesh(),
        compiler_params=_SC_CP,
        scratch_types=[
            pltpu.VMEM((bsz,), jnp.int32),
            pltpu.VMEM((bsz,), jnp.int32),
            pltpu.VMEM((bsz, d), jnp.float32),
            pltpu.VMEM((bsz, d), jnp.float32),
            pltpu.VMEM_SHARED((npad, d), jnp.float32),
        ],
    )
    def k(fp_ref, fn_ref, a0_ref, a1_ref, out_ref,
          i0_v, i1_v, f0_v, f1_v, acc):
        cid = lax.axis_index("c")
        sid = lax.axis_index("s")
        wid = sid * NC + cid

        def zrow(j, c):
            f0_v[j] = jnp.zeros((d,), jnp.float32)
            return c

        lax.fori_loop(0, bsz, zrow, 0)

        def zc(q, c):
            pltpu.sync_copy(f0_v, acc.at[pl.ds(sid * rps + q * bsz, bsz)])
            return c

        lax.fori_loop(0, nchunk, zc, 0)
        plsc.subcore_barrier()

        def body(i, c):
            base = (wid * nblk + i) * bsz
            pltpu.sync_copy(a0_ref.at[pl.ds(base, bsz)], i0_v)
            pltpu.sync_copy(a1_ref.at[pl.ds(base, bsz)], i1_v)
            pltpu.sync_copy(fp_ref.at[pl.ds(base, bsz)], f0_v)
            pltpu.sync_copy(fn_ref.at[pl.ds(base, bsz)], f1_v)
            pltpu.sync_copy(f0_v, acc.at[i0_v], add=True)
            pltpu.sync_copy(f1_v, acc.at[i1_v], add=True)
            return c

        lax.fori_loop(0, nblk, body, 0)
        plsc.subcore_barrier()

        def dump(q, c):
            ro = sid * rps + q * bsz
            pltpu.sync_copy(acc.at[pl.ds(ro, bsz)], f0_v)
            pltpu.sync_copy(f0_v, out_ref.at[cid, pl.ds(ro, bsz)])
            return c

        lax.fori_loop(0, nchunk, dump, 0)

    return k(fpos, fneg, a0, a1)


_BLK = 512


def _row_spec(blk, d):
    return pl.BlockSpec((blk, d), lambda i: (i, 0))


def _full_spec(shape):
    nd = len(shape)
    return pl.BlockSpec(shape, lambda i: (0,) * nd)


def _tc_params():
    return pltpu.CompilerParams(dimension_semantics=("parallel",))


def _tc_geom(dvec, w1, b1, w2, b2, offs, width):
    rows, dcoord = dvec.shape
    grid = rows // _BLK

    def body(dv_ref, offs_ref, w1_ref, b1_ref, w2_ref, b2_ref, e_ref, adj_ref):
        dv = dv_ref[...]
        d2 = jnp.sum(dv * dv, axis=1, keepdims=True)
        dis = jnp.sqrt(d2)
        adj_ref[...] = dv / dis
        x = (dis - offs_ref[...]) / width
        e0 = jnp.exp(-0.5 * x * x)
        h = _ssp(jnp.dot(e0, w1_ref[...], preferred_element_type=jnp.float32)
                 + b1_ref[...])
        e_ref[...] = (jnp.dot(h, w2_ref[...], preferred_element_type=jnp.float32)
                      + b2_ref[...])

    return pl.pallas_call(
        body,
        grid=(grid,),
        in_specs=[
            _row_spec(_BLK, dcoord),
            _full_spec(offs.shape),
            _full_spec(w1.shape),
            _full_spec(b1.shape),
            _full_spec(w2.shape),
            _full_spec(b2.shape),
        ],
        out_specs=[_row_spec(_BLK, 128), _row_spec(_BLK, dcoord)],
        out_shape=[
            jax.ShapeDtypeStruct((rows, 128), jnp.float32),
            jax.ShapeDtypeStruct((rows, dcoord), jnp.float32),
        ],
        compiler_params=_tc_params(),
    )(dvec, offs, w1, b1, w2, b2)


def _tc_mlp(x, p1, p2, res=None, drp_in=False):
    rows = x.shape[1] if drp_in else x.shape[0]
    grid = rows // _BLK
    w1, b1 = p1[0], p1[1].reshape(1, -1)
    w2, b2 = p2[0], p2[1].reshape(1, -1)
    dout = w2.shape[1]

    nsum = x.shape[0] if drp_in else 0

    def body(*refs):
        refs = list(refs)
        x_ref = refs.pop(0)
        w1_ref, b1_ref, w2_ref, b2_ref = refs[:4]
        refs = refs[4:]
        r_ref = refs.pop(0) if res is not None else None
        o_ref = refs.pop(0)
        if drp_in:
            xv = x_ref[0]
            for q in range(1, nsum):
                xv = xv + x_ref[q]
        else:
            xv = x_ref[...]
        h = _ssp(jnp.dot(xv, w1_ref[...],
                         preferred_element_type=jnp.float32) + b1_ref[...])
        o = (jnp.dot(h, w2_ref[...], preferred_element_type=jnp.float32)
             + b2_ref[...])
        if res is not None:
            o = o + r_ref[...]
        o_ref[...] = o

    if drp_in:
        x_spec = pl.BlockSpec((nsum, _BLK, x.shape[2]),
                              lambda i: (0, i, 0))
    else:
        x_spec = _row_spec(_BLK, x.shape[1])

    in_specs = [
        x_spec,
        _full_spec(w1.shape),
        _full_spec(b1.shape),
        _full_spec(w2.shape),
        _full_spec(b2.shape),
    ]
    args = [x, w1, b1, w2, b2]
    if res is not None:
        in_specs.append(_row_spec(_BLK, dout))
        args.append(res)

    return pl.pallas_call(
        body,
        grid=(grid,),
        in_specs=in_specs,
        out_specs=_row_spec(_BLK, dout),
        out_shape=jax.ShapeDtypeStruct((rows, dout), jnp.float32),
        compiler_params=_tc_params(),
    )(*args)


def _tc_readout(e, adj, p1, p2):
    rows, dcoord = adj.shape
    grid = rows // _BLK
    w1, b1 = p1[0], p1[1].reshape(1, -1)
    w2, b2 = p2[0], p2[1].reshape(1, -1)

    def body(e_ref, a_ref, w1_ref, b1_ref, w2_ref, b2_ref, fp_ref, fn_ref):
        h = _ssp(jnp.dot(e_ref[...], w1_ref[...],
                         preferred_element_type=jnp.float32) + b1_ref[...])
        val = (jnp.dot(h, w2_ref[...], preferred_element_type=jnp.float32)
               + b2_ref[...])
        fp = val * a_ref[...]
        fp_ref[...] = fp
        fn_ref[...] = -fp

    return pl.pallas_call(
        body,
        grid=(grid,),
        in_specs=[
            _row_spec(_BLK, 128),
            _row_spec(_BLK, dcoord),
            _full_spec(w1.shape),
            _full_spec(b1.shape),
            _full_spec(w2.shape),
            _full_spec(b2.shape),
        ],
        out_specs=[_row_spec(_BLK, dcoord), _row_spec(_BLK, dcoord)],
        out_shape=[
            jax.ShapeDtypeStruct((rows, dcoord), jnp.float32),
            jax.ShapeDtypeStruct((rows, dcoord), jnp.float32),
        ],
        compiler_params=_tc_params(),
    )(e, adj, w1, b1, w2, b2)


def _tc_combine(facc):
    k, rows, d = facc.shape
    grid = rows // _BLK

    def body(f_ref, o_ref):
        o = f_ref[0]
        for q in range(1, k):
            o = o + f_ref[q]
        o_ref[...] = o

    return pl.pallas_call(
        body,
        grid=(grid,),
        in_specs=[pl.BlockSpec((k, _BLK, d), lambda i: (0, i, 0))],
        out_specs=_row_spec(_BLK, d),
        out_shape=jax.ShapeDtypeStruct((rows, d), jnp.float32),
        compiler_params=_tc_params(),
    )(facc)


def kernel(nxyz, nbr_list, params):
    n = nxyz.shape[0]
    e_cnt = nbr_list.shape[0]
    npad = -(-(n + 1) // 2048) * 2048
    nchk = 8
    quant = NW * 128 * nchk
    epad = -(-e_cnt // quant) * quant
    csz = epad // nchk

    z = nxyz[:, 0].astype(jnp.int32)
    xyz = nxyz[:, 1:4].astype(jnp.float32)
    xyz_pad = jnp.zeros((npad, 16), jnp.float32).at[:n, :3].set(xyz)

    a = nbr_list.astype(jnp.int32)
    a0 = jnp.full((epad,), n, jnp.int32).at[:e_cnt].set(a[:, 0])
    a1 = jnp.full((epad,), n, jnp.int32).at[:e_cnt].set(a[:, 1])
    a0c = [a0[i * csz:(i + 1) * csz] for i in range(nchk)]
    a1c = [a1[i * csz:(i + 1) * csz] for i in range(nchk)]
    z_pad = jnp.zeros((npad,), jnp.int32).at[:n].set(z)

    emb = params['emb']
    emb_pad = jnp.zeros((128, emb.shape[1]), jnp.float32).at[:emb.shape[0]].set(emb)

    r = _sc_gather(emb_pad, z_pad, 64)
    dvec = [_sc_dvec(xyz_pad, a0c[i], a1c[i]) for i in range(nchk)]

    offs = jnp.linspace(0.0, _CUTOFF, _NG).astype(jnp.float32)
    width = float(_CUTOFF / (_NG - 1))
    offs = jnp.concatenate([offs, jnp.full((14,), 1e9, jnp.float32)])
    offs = offs.reshape(1, 64)

    efp = params['ef']
    w1 = jnp.zeros((64, efp[0][0].shape[1]), jnp.float32).at[:_NG].set(efp[0][0])
    ea = [_tc_geom(dvec[i], w1, efp[0][1].reshape(1, -1),
                   efp[1][0], efp[1][1].reshape(1, -1), offs, width)
          for i in range(nchk)]
    e = [x[0] for x in ea]
    adj = [x[1] for x in ea]

    for cp in params['convs']:
        rf = _tc_mlp(r, cp['atom_filter'][0], cp['atom_filter'][1])
        ef = [_tc_mlp(e[i], cp['edge_filter'][0], cp['edge_filter'][1])
              for i in range(nchk)]
        sd = [_sc_convmsg(rf, ef[i], a0c[i], a1c[i], npad)
              for i in range(nchk)]
        drp = jnp.concatenate([x[1] for x in sd], axis=0)
        r = _tc_mlp(drp, cp['atom_update'][0], cp['atom_update'][1],
                    res=r, drp_in=True)
        e = [_tc_mlp(sd[i][0], cp['edge_update'][0], cp['edge_update'][1],
                     res=e[i]) for i in range(nchk)]

    facc = []
    for i in range(nchk):
        fpos, fneg = _tc_readout(e[i], adj[i], params['ro'][0], params['ro'][1])
        facc.append(_sc_fscatter(fpos, fneg, a0c[i], a1c[i], npad))
    f_atom = _tc_combine(jnp.concatenate(facc, axis=0))
    return f_atom[:n, :3]


# speedup vs baseline: 1.6547x; 1.0002x over previous
import functools

import jax
import jax.numpy as jnp
from jax import lax
from jax.experimental import pallas as pl
from jax.experimental.pallas import tpu as pltpu
from jax.experimental.pallas import tpu_sc as plsc

NC, NS, LANES = 2, 16, 16
NW = NC * NS

_LN2 = 0.6931471805599453
_CUTOFF = 5.0
_NG = 50
_HW = 64


def _ssp(x):
    return jax.nn.softplus(x) - _LN2


def _sc_mesh():
    return plsc.VectorSubcoreMesh(core_axis_name="c", subcore_axis_name="s")


_SC_CP = pltpu.CompilerParams(use_tc_tiling_on_sc=False)


def _sc_gather(tbl, idx, bsz):
    rows = idx.shape[0]
    d = tbl.shape[1]
    nblk = rows // (NW * bsz)

    @functools.partial(
        pl.kernel,
        out_type=pltpu.HBM((rows, d), jnp.float32),
        mesh=_sc_mesh(),
        scratch_types=[
            pltpu.VMEM((bsz,), jnp.int32),
            pltpu.VMEM((bsz, d), jnp.float32),
            pltpu.SemaphoreType.DMA,
        ],
    )
    def k(tbl_ref, idx_ref, out_ref, idx_v, rows_v, sem):
        wid = lax.axis_index("s") * NC + lax.axis_index("c")

        def body(i, c):
            base = (wid * nblk + i) * bsz
            pltpu.sync_copy(idx_ref.at[pl.ds(base, bsz)], idx_v)
            pltpu.async_copy(tbl_ref.at[idx_v], rows_v, sem).wait()
            pltpu.sync_copy(rows_v, out_ref.at[pl.ds(base, bsz)])
            return c

        lax.fori_loop(0, nblk, body, 0)

    return k(tbl, idx)


def _sc_dvec(xyz_pad, a0, a1):
    rows = a0.shape[0]
    d = xyz_pad.shape[1]
    bsz = 128
    nblk = rows // (NW * bsz)

    @functools.partial(
        pl.kernel,
        out_type=pltpu.HBM((rows, d), jnp.float32),
        mesh=_sc_mesh(),
        compiler_params=_SC_CP,
        scratch_types=[
            pltpu.VMEM((bsz,), jnp.int32),
            pltpu.VMEM((bsz,), jnp.int32),
            pltpu.VMEM((bsz, d), jnp.float32),
            pltpu.VMEM((bsz, d), jnp.float32),
            pltpu.SemaphoreType.DMA,
            pltpu.SemaphoreType.DMA,
        ],
    )
    def k(tbl_ref, a0_ref, a1_ref, out_ref, i0_v, i1_v, r0_v, r1_v, s0, s1):
        wid = lax.axis_index("s") * NC + lax.axis_index("c")

        def body(i, c):
            base = (wid * nblk + i) * bsz
            pltpu.sync_copy(a0_ref.at[pl.ds(base, bsz)], i0_v)
            pltpu.sync_copy(a1_ref.at[pl.ds(base, bsz)], i1_v)
            c0 = pltpu.async_copy(tbl_ref.at[i0_v], r0_v, s0)
            c1 = pltpu.async_copy(tbl_ref.at[i1_v], r1_v, s1)
            c0.wait()
            c1.wait()

            @functools.partial(plsc.parallel_loop, 0, bsz, unroll=4)
            def row(j):
                r0_v[j] = r0_v[j] - r1_v[j]
            pltpu.sync_copy(r0_v, out_ref.at[pl.ds(base, bsz)])
            return c

        lax.fori_loop(0, nblk, body, 0)

    return k(xyz_pad, a0, a1)


def _sc_convmsg(rf, ef, a0, a1, npad):
    rows = a0.shape[0]
    bsz = 32
    d = rf.shape[1]
    nblk = rows // (NW * bsz)
    rps = npad // NS
    nchunk = rps // bsz

    @functools.partial(
        pl.kernel,
        out_type=(
            pltpu.HBM((rows, d), jnp.float32),
            pltpu.HBM((2, npad, d), jnp.float32),
        ),
        mesh=_sc_mesh(),
        compiler_params=_SC_CP,
        scratch_types=[
            pltpu.VMEM((2, bsz), jnp.int32),
            pltpu.VMEM((2, bsz), jnp.int32),
            pltpu.VMEM((2, bsz, d), jnp.float32),
            pltpu.VMEM((2, bsz, d), jnp.float32),
            pltpu.VMEM((2, bsz, d), jnp.float32),
            pltpu.VMEM((2, bsz, d), jnp.float32),
            pltpu.VMEM_SHARED((npad, d), jnp.float32),
            pltpu.SemaphoreType.DMA,
            pltpu.SemaphoreType.DMA,
            pltpu.SemaphoreType.DMA,
            pltpu.SemaphoreType.DMA,
        ],
    )
    def k(rf_ref, ef_ref, a0_ref, a1_ref, s_out, drp_out,
          i0_v, i1_v, ef_v, r0_v, r1_v, s_v, acc, g0, g1, s0, s1):
        cid = lax.axis_index("c")
        sid = lax.axis_index("s")
        wid = sid * NC + cid
        gsem = (g0, g1)
        ssem = (s0, s1)

        @functools.partial(plsc.parallel_loop, 0, bsz, unroll=4)
        def zero_sv(j):
            for kk in range(d // 16):
                s_v[0, j, pl.ds(kk * 16, 16)] = jnp.zeros((16,), jnp.float32)

        def zc(q, c):
            pltpu.sync_copy(s_v.at[0],
                            acc.at[pl.ds(sid * rps + q * bsz, bsz)])
            return c

        lax.fori_loop(0, nchunk, zc, 0)
        plsc.subcore_barrier()

        def issue_in(i, b):
            base = (wid * nblk + i) * bsz
            pltpu.sync_copy(a0_ref.at[pl.ds(base, bsz)], i0_v.at[b])
            pltpu.sync_copy(a1_ref.at[pl.ds(base, bsz)], i1_v.at[b])
            pltpu.async_copy(rf_ref.at[i0_v.at[b]], r0_v.at[b], gsem[b])
            pltpu.async_copy(rf_ref.at[i1_v.at[b]], r1_v.at[b], gsem[b])
            pltpu.async_copy(ef_ref.at[pl.ds(base, bsz)], ef_v.at[b], gsem[b])

        def wait_in(b):
            pltpu.make_async_copy(rf_ref.at[i0_v.at[b]], r0_v.at[b],
                                  gsem[b]).wait()
            pltpu.make_async_copy(rf_ref.at[i1_v.at[b]], r1_v.at[b],
                                  gsem[b]).wait()
            pltpu.make_async_copy(ef_ref.at[pl.ds(0, bsz)], ef_v.at[b],
                                  gsem[b]).wait()

        def compute(b):
            @functools.partial(plsc.parallel_loop, 0, bsz, unroll=4)
            def row(j):
                for kk in range(d // 16):
                    dsl = pl.ds(kk * 16, 16)
                    e = ef_v[b, j, dsl]
                    x0 = r0_v[b, j, dsl] * e
                    x1 = r1_v[b, j, dsl] * e
                    r0_v[b, j, dsl] = x0
                    r1_v[b, j, dsl] = x1
                    s_v[b, j, dsl] = x0 + x1

        def finish(i, b):
            base = (wid * nblk + i) * bsz
            pltpu.sync_copy(r0_v.at[b], acc.at[i1_v.at[b]], add=True)
            pltpu.sync_copy(r1_v.at[b], acc.at[i0_v.at[b]], add=True)
            pltpu.async_copy(s_v.at[b], s_out.at[pl.ds(base, bsz)], ssem[b])

        def wait_s(b):
            pltpu.make_async_copy(s_v.at[b], s_out.at[pl.ds(0, bsz)],
                                  ssem[b]).wait()

        issue_in(0, 0)

        issue_in(1, 1)
        wait_in(0)
        compute(0)
        finish(0, 0)
        issue_in(2, 0)
        wait_in(1)
        compute(1)
        finish(1, 1)

        def body(g, c):
            i = g * 2

            issue_in(i + 1, 1)
            wait_in(0)
            wait_s(0)
            compute(0)
            finish(i, 0)

            @pl.when(g + 1 < nblk // 2)
            def _():
                issue_in(i + 2, 0)

            wait_in(1)
            wait_s(1)
            compute(1)
            finish(i + 1, 1)
            return c

        lax.fori_loop(1, nblk // 2, body, 0)
        wait_s(0)
        wait_s(1)
        plsc.subcore_barrier()

        def dump(q, c):
            ro = sid * rps + q * bsz
            pltpu.sync_copy(acc.at[pl.ds(ro, bsz)], s_v.at[0])
            pltpu.sync_copy(s_v.at[0], drp_out.at[cid, pl.ds(ro, bsz)])
            return c

        lax.fori_loop(0, nchunk, dump, 0)

    return k(rf, ef, a0, a1)


def _sc_fscatter(fpos, fneg, a0, a1, npad):
    rows = a0.shape[0]
    d = fpos.shape[1]
    bsz = 128
    nblk = rows // (NW * bsz)
    rps = npad // NS
    nchunk = rps // bsz

    @functools.partial(
        pl.kernel,
        out_type=pltpu.HBM((2, npad, d), jnp.float32),
        mesh=_sc_mesh(),
        compiler_params=_SC_CP,
        scratch_types=[
            pltpu.VMEM((bsz,), jnp.int32),
            pltpu.VMEM((bsz,), jnp.int32),
            pltpu.VMEM((bsz, d), jnp.float32),
            pltpu.VMEM((bsz, d), jnp.float32),
            pltpu.VMEM_SHARED((npad, d), jnp.float32),
        ],
    )
    def k(fp_ref, fn_ref, a0_ref, a1_ref, out_ref,
          i0_v, i1_v, f0_v, f1_v, acc):
        cid = lax.axis_index("c")
        sid = lax.axis_index("s")
        wid = sid * NC + cid

        def zrow(j, c):
            f0_v[j] = jnp.zeros((d,), jnp.float32)
            return c

        lax.fori_loop(0, bsz, zrow, 0)

        def zc(q, c):
            pltpu.sync_copy(f0_v, acc.at[pl.ds(sid * rps + q * bsz, bsz)])
            return c

        lax.fori_loop(0, nchunk, zc, 0)
        plsc.subcore_barrier()

        def body(i, c):
            base = (wid * nblk + i) * bsz
            pltpu.sync_copy(a0_ref.at[pl.ds(base, bsz)], i0_v)
            pltpu.sync_copy(a1_ref.at[pl.ds(base, bsz)], i1_v)
            pltpu.sync_copy(fp_ref.at[pl.ds(base, bsz)], f0_v)
            pltpu.sync_copy(fn_ref.at[pl.ds(base, bsz)], f1_v)
            pltpu.sync_copy(f0_v, acc.at[i0_v], add=True)
            pltpu.sync_copy(f1_v, acc.at[i1_v], add=True)
            return c

        lax.fori_loop(0, nblk, body, 0)
        plsc.subcore_barrier()

        def dump(q, c):
            ro = sid * rps + q * bsz
            pltpu.sync_copy(acc.at[pl.ds(ro, bsz)], f0_v)
            pltpu.sync_copy(f0_v, out_ref.at[cid, pl.ds(ro, bsz)])
            return c

        lax.fori_loop(0, nchunk, dump, 0)

    return k(fpos, fneg, a0, a1)


_BLK = 512


def _row_spec(blk, d):
    return pl.BlockSpec((blk, d), lambda i: (i, 0))


def _full_spec(shape):
    nd = len(shape)
    return pl.BlockSpec(shape, lambda i: (0,) * nd)


def _tc_params():
    return pltpu.CompilerParams(dimension_semantics=("parallel",))


def _tc_geom(dvec, w1, b1, w2, b2, offs, width):
    rows, dcoord = dvec.shape
    grid = rows // _BLK

    def body(dv_ref, offs_ref, w1_ref, b1_ref, w2_ref, b2_ref, e_ref, adj_ref):
        dv = dv_ref[...]
        d2 = jnp.sum(dv * dv, axis=1, keepdims=True)
        dis = jnp.sqrt(d2)
        adj_ref[...] = dv / dis
        x = (dis - offs_ref[...]) / width
        e0 = jnp.exp(-0.5 * x * x)
        h = _ssp(jnp.dot(e0, w1_ref[...], preferred_element_type=jnp.float32)
                 + b1_ref[...])
        e_ref[...] = (jnp.dot(h, w2_ref[...], preferred_element_type=jnp.float32)
                      + b2_ref[...])

    return pl.pallas_call(
        body,
        grid=(grid,),
        in_specs=[
            _row_spec(_BLK, dcoord),
            _full_spec(offs.shape),
            _full_spec(w1.shape),
            _full_spec(b1.shape),
            _full_spec(w2.shape),
            _full_spec(b2.shape),
        ],
        out_specs=[_row_spec(_BLK, 128), _row_spec(_BLK, dcoord)],
        out_shape=[
            jax.ShapeDtypeStruct((rows, 128), jnp.float32),
            jax.ShapeDtypeStruct((rows, dcoord), jnp.float32),
        ],
        compiler_params=_tc_params(),
    )(dvec, offs, w1, b1, w2, b2)


def _tc_mlp(x, p1, p2, res=None, drp_in=False):
    rows = x.shape[1] if drp_in else x.shape[0]
    grid = rows // _BLK
    w1, b1 = p1[0], p1[1].reshape(1, -1)
    w2, b2 = p2[0], p2[1].reshape(1, -1)
    dout = w2.shape[1]

    nsum = x.shape[0] if drp_in else 0

    def body(*refs):
        refs = list(refs)
        x_ref = refs.pop(0)
        w1_ref, b1_ref, w2_ref, b2_ref = refs[:4]
        refs = refs[4:]
        r_ref = refs.pop(0) if res is not None else None
        o_ref = refs.pop(0)
        if drp_in:
            xv = x_ref[0]
            for q in range(1, nsum):
                xv = xv + x_ref[q]
        else:
            xv = x_ref[...]
        h = _ssp(jnp.dot(xv, w1_ref[...],
                         preferred_element_type=jnp.float32) + b1_ref[...])
        o = (jnp.dot(h, w2_ref[...], preferred_element_type=jnp.float32)
             + b2_ref[...])
        if res is not None:
            o = o + r_ref[...]
        o_ref[...] = o

    if drp_in:
        x_spec = pl.BlockSpec((nsum, _BLK, x.shape[2]),
                              lambda i: (0, i, 0))
    else:
        x_spec = _row_spec(_BLK, x.shape[1])

    in_specs = [
        x_spec,
        _full_spec(w1.shape),
        _full_spec(b1.shape),
        _full_spec(w2.shape),
        _full_spec(b2.shape),
    ]
    args = [x, w1, b1, w2, b2]
    if res is not None:
        in_specs.append(_row_spec(_BLK, dout))
        args.append(res)

    return pl.pallas_call(
        body,
        grid=(grid,),
        in_specs=in_specs,
        out_specs=_row_spec(_BLK, dout),
        out_shape=jax.ShapeDtypeStruct((rows, dout), jnp.float32),
        compiler_params=_tc_params(),
    )(*args)


def _tc_readout(e, adj, p1, p2):
    rows, dcoord = adj.shape
    grid = rows // _BLK
    w1, b1 = p1[0], p1[1].reshape(1, -1)
    w2, b2 = p2[0], p2[1].reshape(1, -1)

    def body(e_ref, a_ref, w1_ref, b1_ref, w2_ref, b2_ref, fp_ref, fn_ref):
        h = _ssp(jnp.dot(e_ref[...], w1_ref[...],
                         preferred_element_type=jnp.float32) + b1_ref[...])
        val = (jnp.dot(h, w2_ref[...], preferred_element_type=jnp.float32)
               + b2_ref[...])
        fp = val * a_ref[...]
        fp_ref[...] = fp
        fn_ref[...] = -fp

    return pl.pallas_call(
        body,
        grid=(grid,),
        in_specs=[
            _row_spec(_BLK, 128),
            _row_spec(_BLK, dcoord),
            _full_spec(w1.shape),
            _full_spec(b1.shape),
            _full_spec(w2.shape),
            _full_spec(b2.shape),
        ],
        out_specs=[_row_spec(_BLK, dcoord), _row_spec(_BLK, dcoord)],
        out_shape=[
            jax.ShapeDtypeStruct((rows, dcoord), jnp.float32),
            jax.ShapeDtypeStruct((rows, dcoord), jnp.float32),
        ],
        compiler_params=_tc_params(),
    )(e, adj, w1, b1, w2, b2)


def _tc_combine(facc):
    k, rows, d = facc.shape
    grid = rows // _BLK

    def body(f_ref, o_ref):
        o = f_ref[0]
        for q in range(1, k):
            o = o + f_ref[q]
        o_ref[...] = o

    return pl.pallas_call(
        body,
        grid=(grid,),
        in_specs=[pl.BlockSpec((k, _BLK, d), lambda i: (0, i, 0))],
        out_specs=_row_spec(_BLK, d),
        out_shape=jax.ShapeDtypeStruct((rows, d), jnp.float32),
        compiler_params=_tc_params(),
    )(facc)


def kernel(nxyz, nbr_list, params):
    n = nxyz.shape[0]
    e_cnt = nbr_list.shape[0]
    npad = -(-(n + 1) // 2048) * 2048
    nchk = 8
    quant = NW * 128 * nchk
    epad = -(-e_cnt // quant) * quant
    csz = epad // nchk

    z = nxyz[:, 0].astype(jnp.int32)
    xyz = nxyz[:, 1:4].astype(jnp.float32)
    xyz_pad = jnp.zeros((npad, 16), jnp.float32).at[:n, :3].set(xyz)

    a = nbr_list.astype(jnp.int32)
    a0 = jnp.full((epad,), n, jnp.int32).at[:e_cnt].set(a[:, 0])
    a1 = jnp.full((epad,), n, jnp.int32).at[:e_cnt].set(a[:, 1])
    a0c = [a0[i * csz:(i + 1) * csz] for i in range(nchk)]
    a1c = [a1[i * csz:(i + 1) * csz] for i in range(nchk)]
    z_pad = jnp.zeros((npad,), jnp.int32).at[:n].set(z)

    emb = params['emb']
    emb_pad = jnp.zeros((128, emb.shape[1]), jnp.float32).at[:emb.shape[0]].set(emb)

    r = _sc_gather(emb_pad, z_pad, 64)
    dvec = [_sc_dvec(xyz_pad, a0c[i], a1c[i]) for i in range(nchk)]

    offs = jnp.linspace(0.0, _CUTOFF, _NG).astype(jnp.float32)
    width = float(_CUTOFF / (_NG - 1))
    offs = jnp.concatenate([offs, jnp.full((14,), 1e9, jnp.float32)])
    offs = offs.reshape(1, 64)

    efp = params['ef']
    w1 = jnp.zeros((64, efp[0][0].shape[1]), jnp.float32).at[:_NG].set(efp[0][0])
    ea = [_tc_geom(dvec[i], w1, efp[0][1].reshape(1, -1),
                   efp[1][0], efp[1][1].reshape(1, -1), offs, width)
          for i in range(nchk)]
    e = [x[0] for x in ea]
    adj = [x[1] for x in ea]

    for cp in params['convs']:
        rf = _tc_mlp(r, cp['atom_filter'][0], cp['atom_filter'][1])
        ef = [_tc_mlp(e[i], cp['edge_filter'][0], cp['edge_filter'][1])
              for i in range(nchk)]
        sd = [_sc_convmsg(rf, ef[i], a0c[i], a1c[i], npad)
              for i in range(nchk)]
        drp = jnp.concatenate([x[1] for x in sd], axis=0)
        r = _tc_mlp(drp, cp['atom_update'][0], cp['atom_update'][1],
                    res=r, drp_in=True)
        e = [_tc_mlp(sd[i][0], cp['edge_update'][0], cp['edge_update'][1],
                     res=e[i]) for i in range(nchk)]

    facc = []
    for i in range(nchk):
        fpos, fneg = _tc_readout(e[i], adj[i], params['ro'][0], params['ro'][1])
        facc.append(_sc_fscatter(fpos, fneg, a0c[i], a1c[i], npad))
    f_atom = _tc_combine(jnp.concatenate(facc, axis=0))
    return f_atom[:n, :3]
